# baseline (speedup 1.0000x reference)
import functools

import jax
import jax.numpy as jnp
import numpy as np
from jax import lax
from jax.experimental import pallas as pl
from jax.experimental.pallas import tpu as pltpu
from jax.experimental.pallas import tpu_sc as plsc

_B, _N, _C = 2, 2048, 768
_H = 12
_Dh = _C // _H
_R = 256
_HALF = _N // 2
_NM = _N - _R

_NC = 2
_NS = 16
_NW = _NC * _NS
_MPW = (_B * _NM) // _NW
_MCH = _MPW // 2
_UPW = (_B * _N) // _NW
_PAD = 8
_ZROW = _B * _NM


def _match_kernel(xe_ref, xo_ref, wk_ref, bs_ref, bb_ref):
    a = jnp.dot(xe_ref[0], wk_ref[...], preferred_element_type=jnp.float32)
    b = jnp.dot(xo_ref[0], wk_ref[...], preferred_element_type=jnp.float32)
    an = a / jnp.maximum(
        jnp.sqrt(jnp.sum(a * a, axis=1, keepdims=True)), 1e-12)
    bn = b / jnp.maximum(
        jnp.sqrt(jnp.sum(b * b, axis=1, keepdims=True)), 1e-12)
    scores = jax.lax.dot_general(
        an, bn, (((1,), (1,)), ((), ())),
        preferred_element_type=jnp.float32)
    m = jnp.max(scores, axis=1, keepdims=True)
    cols = jax.lax.broadcasted_iota(jnp.int32, scores.shape, 1)
    cand = jnp.where(scores == m, cols, jnp.int32(_HALF))
    bs_ref[0, 0, :] = m[:, 0]
    bb_ref[0, 0, :] = jnp.min(cand, axis=1)


def _match(x_even, x_odd, wk):
    bs, bb = pl.pallas_call(
        _match_kernel,
        grid=(_B,),
        in_specs=[
            pl.BlockSpec((1, _HALF, _C), lambda i: (i, 0, 0)),
            pl.BlockSpec((1, _HALF, _C), lambda i: (i, 0, 0)),
            pl.BlockSpec((_C, _C), lambda i: (0, 0)),
        ],
        out_specs=[
            pl.BlockSpec((1, 1, _HALF), lambda i: (i, 0, 0)),
            pl.BlockSpec((1, 1, _HALF), lambda i: (i, 0, 0)),
        ],
        out_shape=[
            jax.ShapeDtypeStruct((_B, 1, _HALF), jnp.float32),
            jax.ShapeDtypeStruct((_B, 1, _HALF), jnp.int32),
        ],
    )(x_even, x_odd, wk)
    return bs[:, 0, :], bb[:, 0, :]


_HP = 2
_HB = _H // _HP


def _qkv_kernel(xm_ref, wq_ref, wk_ref, wv_ref, cos_ref, sin_ref, s_ref,
                q_ref, k_ref, v_ref):
    x = xm_ref[0]
    cos = cos_ref[...]
    sin = sin_ref[...]
    s_mat = s_ref[...]

    q = jnp.dot(x, wq_ref[...], preferred_element_type=jnp.float32)
    q_rot = jnp.dot(q, s_mat, preferred_element_type=jnp.float32)
    qr = q * cos + q_rot * sin
    q_ref[0, 0] = qr[:, :_Dh]
    q_ref[0, 1] = qr[:, _Dh:]

    k = jnp.dot(x, wk_ref[...], preferred_element_type=jnp.float32)
    k_rot = jnp.dot(k, s_mat, preferred_element_type=jnp.float32)
    kr = k * cos + k_rot * sin
    k_ref[0, 0] = kr[:, :_Dh]
    k_ref[0, 1] = kr[:, _Dh:]

    v = jnp.dot(x, wv_ref[...], preferred_element_type=jnp.float32)
    v_ref[0, 0] = v[:, :_Dh]
    v_ref[0, 1] = v[:, _Dh:]


def _qkv(x_m, wq, wk, wv, cos_i, sin_i, s_mat):
    wcols = _HP * _Dh
    return pl.pallas_call(
        _qkv_kernel,
        grid=(_B, _HB),
        in_specs=[
            pl.BlockSpec((1, _NM, _C), lambda b, h: (b, 0, 0)),
            pl.BlockSpec((_C, wcols), lambda b, h: (0, h)),
            pl.BlockSpec((_C, wcols), lambda b, h: (0, h)),
            pl.BlockSpec((_C, wcols), lambda b, h: (0, h)),
            pl.BlockSpec((_NM, wcols), lambda b, h: (0, 0)),
            pl.BlockSpec((_NM, wcols), lambda b, h: (0, 0)),
            pl.BlockSpec((wcols, wcols), lambda b, h: (0, 0)),
        ],
        out_specs=[
            pl.BlockSpec((1, _HP, _NM, _Dh), lambda b, h: (b, h, 0, 0)),
            pl.BlockSpec((1, _HP, _NM, _Dh), lambda b, h: (b, h, 0, 0)),
            pl.BlockSpec((1, _HP, _NM, _Dh), lambda b, h: (b, h, 0, 0)),
        ],
        out_shape=[
            jax.ShapeDtypeStruct((_B, _H, _NM, _Dh), jnp.float32),
            jax.ShapeDtypeStruct((_B, _H, _NM, _Dh), jnp.float32),
            jax.ShapeDtypeStruct((_B, _H, _NM, _Dh), jnp.float32),
        ],
    )(x_m, wq, wk, wv, cos_i, sin_i, s_mat)


def _attn_kernel(q_ref, k_ref, v_ref, o_ref):
    q = q_ref[0, 0]
    k = k_ref[0, 0]
    v = v_ref[0, 0]
    s = jax.lax.dot_general(
        q, k, (((1,), (1,)), ((), ())),
        preferred_element_type=jnp.float32)
    s = s * jnp.float32(1.0 / np.sqrt(_Dh))
    rows = jax.lax.broadcasted_iota(jnp.int32, s.shape, 0)
    cols = jax.lax.broadcasted_iota(jnp.int32, s.shape, 1)
    s = jnp.where(rows >= cols, s, jnp.float32(-1e9))
    m = jnp.max(s, axis=1, keepdims=True)
    e = jnp.exp(s - m)
    p = e / jnp.sum(e, axis=1, keepdims=True)
    o_ref[0, 0] = jnp.dot(p, v, preferred_element_type=jnp.float32)


def _attention(q, k, v):
    return pl.pallas_call(
        _attn_kernel,
        grid=(_B, _H),
        in_specs=[
            pl.BlockSpec((1, 1, _NM, _Dh), lambda b, h: (b, h, 0, 0)),
            pl.BlockSpec((1, 1, _NM, _Dh), lambda b, h: (b, h, 0, 0)),
            pl.BlockSpec((1, 1, _NM, _Dh), lambda b, h: (b, h, 0, 0)),
        ],
        out_specs=pl.BlockSpec((1, 1, _NM, _Dh), lambda b, h: (b, h, 0, 0)),
        out_shape=jax.ShapeDtypeStruct((_B, _H, _NM, _Dh), jnp.float32),
    )(q, k, v)


def _proj_kernel(a_ref, wo_ref, o_ref):
    acc = jnp.zeros((_NM, _C), dtype=jnp.float32)
    for h in range(_H):
        acc = acc + jnp.dot(
            a_ref[0, h], wo_ref[h * _Dh:(h + 1) * _Dh, :],
            preferred_element_type=jnp.float32)
    o_ref[0] = acc


def _out_proj(att, wo):
    return pl.pallas_call(
        _proj_kernel,
        grid=(_B,),
        in_specs=[
            pl.BlockSpec((1, _H, _NM, _Dh), lambda b: (b, 0, 0, 0)),
            pl.BlockSpec((_C, _C), lambda b: (0, 0)),
        ],
        out_specs=pl.BlockSpec((1, _NM, _C), lambda b: (b, 0, 0)),
        out_shape=jax.ShapeDtypeStruct((_B, _NM, _C), jnp.float32),
    )(att, wo)


_sc_mesh = plsc.VectorSubcoreMesh(
    core_axis_name="c", subcore_axis_name="s", num_cores=_NC)


@functools.partial(
    pl.kernel, mesh=_sc_mesh,
    out_type=jax.ShapeDtypeStruct((_B * _NM, _C), jnp.float32),
    scratch_types=[
        pltpu.VMEM((_MCH,), jnp.int32),
        pltpu.VMEM((_MCH,), jnp.int32),
        pltpu.VMEM((_MCH, _C), jnp.float32),
        pltpu.VMEM((_MCH, _C), jnp.float32),
        pltpu.SemaphoreType.DMA,
        pltpu.SemaphoreType.DMA,
    ])
def _sc_merge(x_hbm, i1_hbm, i2_hbm, out_hbm, i1_v, i2_v, r1_v, r2_v, s1, s2):
    wid = lax.axis_index("s") * _NC + lax.axis_index("c")
    base = wid * _MPW
    for c in range(_MPW // _MCH):
        co = base + c * _MCH
        pltpu.sync_copy(i1_hbm.at[pl.ds(co, _MCH)], i1_v)
        pltpu.sync_copy(i2_hbm.at[pl.ds(co, _MCH)], i2_v)
        cp1 = pltpu.async_copy(x_hbm.at[i1_v], r1_v, s1)
        cp2 = pltpu.async_copy(x_hbm.at[i2_v], r2_v, s2)
        cp1.wait()
        cp2.wait()

        def row_body(i, _):
            def grp_body(j, _):
                sl = pl.ds(j * 16, 16)
                r1_v[i, sl] = (r1_v[i, sl] + r2_v[i, sl]) * 0.5
                return 0
            return lax.fori_loop(0, _C // 16, grp_body, 0)

        lax.fori_loop(0, _MCH, row_body, 0)
        pltpu.sync_copy(r1_v, out_hbm.at[pl.ds(co, _MCH)])


@functools.partial(
    pl.kernel, mesh=_sc_mesh,
    out_type=jax.ShapeDtypeStruct((_B * _N, _C), jnp.float32),
    scratch_types=[
        pltpu.VMEM((_UPW,), jnp.int32),
        pltpu.VMEM((_UPW, _C), jnp.float32),
        pltpu.SemaphoreType.DMA,
    ])
def _sc_unmerge(tbl_hbm, src_hbm, out_hbm, idx_v, rows_v, sem):
    wid = lax.axis_index("s") * _NC + lax.axis_index("c")
    base = wid * _UPW
    pltpu.sync_copy(src_hbm.at[pl.ds(base, _UPW)], idx_v)
    pltpu.async_copy(tbl_hbm.at[idx_v], rows_v, sem).wait()
    pltpu.sync_copy(rows_v, out_hbm.at[pl.ds(base, _UPW)])


def _select_pairs(best_s, best_b):
    bi = jnp.arange(_B, dtype=jnp.int32)[:, None]
    order = jnp.argsort(-best_s, axis=1).astype(jnp.int32)
    bb_ord = jnp.take_along_axis(best_b, order, axis=1)
    pos = jnp.arange(_HALF, dtype=jnp.int32)[None, :]
    firstpos = jnp.full((_B, _HALF), _HALF, dtype=jnp.int32)
    firstpos = firstpos.at[bi, bb_ord].min(
        jnp.broadcast_to(pos, (_B, _HALF)))
    take0 = jnp.take_along_axis(firstpos, bb_ord, axis=1) == pos
    c0 = jnp.cumsum(take0.astype(jnp.int32), axis=1) - take0.astype(jnp.int32)
    take = jnp.logical_and(take0, c0 < _R)
    cnt = jnp.sum(take.astype(jnp.int32), axis=1)
    sel = jnp.argsort(jnp.logical_not(take).astype(jnp.int32),
                      axis=1).astype(jnp.int32)[:, :_R]
    ca = jnp.take_along_axis(order, sel, axis=1)
    slots = jnp.arange(_R, dtype=jnp.int32)[None, :]
    ca = jnp.where(slots < cnt[:, None], ca, ca[:, :1])
    cb = jnp.take_along_axis(best_b, ca, axis=1)
    ga = ca * 2
    gb = cb * 2 + 1
    rm = jnp.zeros((_B, _N), dtype=bool).at[bi, gb].set(True)
    keep = jnp.argsort(rm.astype(jnp.int32), axis=1).astype(jnp.int32)[:, :_NM]
    return ga, gb, keep, rm


def _build_s_mat():
    s = np.zeros((_HP * _Dh, _HP * _Dh), dtype=np.float32)
    for h in range(_HP):
        o = h * _Dh
        for i in range(_Dh // 2):
            s[o + 2 * i + 1, o + 2 * i] = -1.0
            s[o + 2 * i, o + 2 * i + 1] = 1.0
    return jnp.asarray(s)


@jax.jit
def _run(x, freqs_cis, wq, wk, wv, wo):
    bi = jnp.arange(_B, dtype=jnp.int32)[:, None]

    x_even = x[:, 0::2, :]
    x_odd = x[:, 1::2, :]
    best_s, best_b = _match(x_even, x_odd, wk)
    ga, gb, keep, rm = _select_pairs(best_s, best_b)

    pmerge = jnp.broadcast_to(
        jnp.arange(_N, dtype=jnp.int32)[None, :], (_B, _N))
    pmerge = pmerge.at[bi, ga].set(gb)
    idx2 = jnp.take_along_axis(pmerge, keep, axis=1)
    i1f = (keep + bi * _N).reshape(-1)
    i2f = (idx2 + bi * _N).reshape(-1)
    x_m = _sc_merge(x.reshape(_B * _N, _C), i1f, i2f).reshape(_B, _NM, _C)

    cos = freqs_cis[:_NM, :, 0]
    sin = freqs_cis[:_NM, :, 1]
    cos_i = jnp.tile(jnp.repeat(cos, 2, axis=1), (1, _HP))
    sin_i = jnp.tile(jnp.repeat(sin, 2, axis=1), (1, _HP))
    s_mat = _build_s_mat()

    q, k, v = _qkv(x_m, wq, wk, wv, cos_i, sin_i, s_mat)
    att = _attention(q, k, v)
    out_m = _out_proj(att, wo)

    rows = jnp.zeros((_B, _N), dtype=jnp.int32).at[bi, keep].set(
        jnp.broadcast_to(jnp.arange(_NM, dtype=jnp.int32)[None, :],
                         (_B, _NM)))
    in_keep = jnp.zeros((_B, _N), dtype=bool).at[bi, keep].set(True)
    pb = jnp.broadcast_to(
        jnp.arange(_N, dtype=jnp.int32)[None, :], (_B, _N))
    pb = pb.at[bi, gb].set(ga)
    src = jnp.where(rm, jnp.take_along_axis(rows, pb, axis=1), rows)
    valid = jnp.logical_or(
        in_keep,
        jnp.logical_and(rm, jnp.take_along_axis(in_keep, pb, axis=1)))
    srcf = jnp.where(valid, src + bi * _NM, _ZROW).reshape(-1)
    tbl = jnp.concatenate(
        [out_m.reshape(_B * _NM, _C),
         jnp.zeros((_PAD, _C), jnp.float32)], axis=0)
    out = _sc_unmerge(tbl, srcf).reshape(_B, _N, _C)
    return out, k, v


def kernel(x, freqs_cis, Wq, Wk, Wv, Wo):
    return _run(x, freqs_cis, Wq, Wk, Wv, Wo)


# speedup vs baseline: 1.0028x; 1.0028x over previous
import functools

import jax
import jax.numpy as jnp
import numpy as np
from jax import lax
from jax.experimental import pallas as pl
from jax.experimental.pallas import tpu as pltpu
from jax.experimental.pallas import tpu_sc as plsc

_B, _N, _C = 2, 2048, 768
_H = 12
_Dh = _C // _H
_R = 256
_HALF = _N // 2
_NM = _N - _R

_NC = 2
_NS = 16
_NW = _NC * _NS
_MPW = (_B * _NM) // _NW
_MCH = _MPW // 2
_UPW = (_B * _N) // _NW
_PAD = 8
_ZROW = _B * _NM


def _match_kernel(xe_ref, xo_ref, wk_ref, bs_ref, bb_ref):
    a = jnp.dot(xe_ref[0], wk_ref[...], preferred_element_type=jnp.float32)
    b = jnp.dot(xo_ref[0], wk_ref[...], preferred_element_type=jnp.float32)
    an = a / jnp.maximum(
        jnp.sqrt(jnp.sum(a * a, axis=1, keepdims=True)), 1e-12)
    bn = b / jnp.maximum(
        jnp.sqrt(jnp.sum(b * b, axis=1, keepdims=True)), 1e-12)
    scores = jax.lax.dot_general(
        an, bn, (((1,), (1,)), ((), ())),
        preferred_element_type=jnp.float32)
    m = jnp.max(scores, axis=1, keepdims=True)
    cols = jax.lax.broadcasted_iota(jnp.int32, scores.shape, 1)
    cand = jnp.where(scores == m, cols, jnp.int32(_HALF))
    bs_ref[0, 0, :] = m[:, 0]
    bb_ref[0, 0, :] = jnp.min(cand, axis=1)


def _match(x_even, x_odd, wk):
    bs, bb = pl.pallas_call(
        _match_kernel,
        grid=(_B,),
        in_specs=[
            pl.BlockSpec((1, _HALF, _C), lambda i: (i, 0, 0)),
            pl.BlockSpec((1, _HALF, _C), lambda i: (i, 0, 0)),
            pl.BlockSpec((_C, _C), lambda i: (0, 0)),
        ],
        out_specs=[
            pl.BlockSpec((1, 1, _HALF), lambda i: (i, 0, 0)),
            pl.BlockSpec((1, 1, _HALF), lambda i: (i, 0, 0)),
        ],
        out_shape=[
            jax.ShapeDtypeStruct((_B, 1, _HALF), jnp.float32),
            jax.ShapeDtypeStruct((_B, 1, _HALF), jnp.int32),
        ],
    )(x_even, x_odd, wk)
    return bs[:, 0, :], bb[:, 0, :]


_HP = 2
_HB = _H // _HP


def _qkv_kernel(xm_ref, wq_ref, wk_ref, wv_ref, cos_ref, sin_ref, s_ref,
                q_ref, k_ref, v_ref):
    x = xm_ref[0]
    cos = cos_ref[...]
    sin = sin_ref[...]
    s_mat = s_ref[...]

    q = jnp.dot(x, wq_ref[...], preferred_element_type=jnp.float32)
    q_rot = jnp.dot(q, s_mat, preferred_element_type=jnp.float32)
    qr = q * cos + q_rot * sin
    q_ref[0, 0] = qr[:, :_Dh]
    q_ref[0, 1] = qr[:, _Dh:]

    k = jnp.dot(x, wk_ref[...], preferred_element_type=jnp.float32)
    k_rot = jnp.dot(k, s_mat, preferred_element_type=jnp.float32)
    kr = k * cos + k_rot * sin
    k_ref[0, 0] = kr[:, :_Dh]
    k_ref[0, 1] = kr[:, _Dh:]

    v = jnp.dot(x, wv_ref[...], preferred_element_type=jnp.float32)
    v_ref[0, 0] = v[:, :_Dh]
    v_ref[0, 1] = v[:, _Dh:]


def _qkv(x_m, wq, wk, wv, cos_i, sin_i, s_mat):
    wcols = _HP * _Dh
    return pl.pallas_call(
        _qkv_kernel,
        grid=(_B, _HB),
        in_specs=[
            pl.BlockSpec((1, _NM, _C), lambda b, h: (b, 0, 0)),
            pl.BlockSpec((_C, wcols), lambda b, h: (0, h)),
            pl.BlockSpec((_C, wcols), lambda b, h: (0, h)),
            pl.BlockSpec((_C, wcols), lambda b, h: (0, h)),
            pl.BlockSpec((_NM, wcols), lambda b, h: (0, 0)),
            pl.BlockSpec((_NM, wcols), lambda b, h: (0, 0)),
            pl.BlockSpec((wcols, wcols), lambda b, h: (0, 0)),
        ],
        out_specs=[
            pl.BlockSpec((1, _HP, _NM, _Dh), lambda b, h: (b, h, 0, 0)),
            pl.BlockSpec((1, _HP, _NM, _Dh), lambda b, h: (b, h, 0, 0)),
            pl.BlockSpec((1, _HP, _NM, _Dh), lambda b, h: (b, h, 0, 0)),
        ],
        out_shape=[
            jax.ShapeDtypeStruct((_B, _H, _NM, _Dh), jnp.float32),
            jax.ShapeDtypeStruct((_B, _H, _NM, _Dh), jnp.float32),
            jax.ShapeDtypeStruct((_B, _H, _NM, _Dh), jnp.float32),
        ],
    )(x_m, wq, wk, wv, cos_i, sin_i, s_mat)


def _attn_kernel(q_ref, k_ref, v_ref, o_ref):
    q = q_ref[0, 0].astype(jnp.bfloat16)
    k = k_ref[0, 0].astype(jnp.bfloat16)
    v = v_ref[0, 0].astype(jnp.bfloat16)
    s = jax.lax.dot_general(
        q, k, (((1,), (1,)), ((), ())),
        preferred_element_type=jnp.float32)
    s = s * jnp.float32(1.0 / np.sqrt(_Dh))
    rows = jax.lax.broadcasted_iota(jnp.int32, s.shape, 0)
    cols = jax.lax.broadcasted_iota(jnp.int32, s.shape, 1)
    s = jnp.where(rows >= cols, s, jnp.float32(-1e9))
    m = jnp.max(s, axis=1, keepdims=True)
    e = jnp.exp(s - m)
    p = (e / jnp.sum(e, axis=1, keepdims=True)).astype(jnp.bfloat16)
    o_ref[0, 0] = jnp.dot(p, v, preferred_element_type=jnp.float32)


def _attention(q, k, v):
    return pl.pallas_call(
        _attn_kernel,
        grid=(_B, _H),
        in_specs=[
            pl.BlockSpec((1, 1, _NM, _Dh), lambda b, h: (b, h, 0, 0)),
            pl.BlockSpec((1, 1, _NM, _Dh), lambda b, h: (b, h, 0, 0)),
            pl.BlockSpec((1, 1, _NM, _Dh), lambda b, h: (b, h, 0, 0)),
        ],
        out_specs=pl.BlockSpec((1, 1, _NM, _Dh), lambda b, h: (b, h, 0, 0)),
        out_shape=jax.ShapeDtypeStruct((_B, _H, _NM, _Dh), jnp.float32),
    )(q, k, v)


def _proj_kernel(a_ref, wo_ref, o_ref):
    acc = jnp.zeros((_NM, _C), dtype=jnp.float32)
    for h in range(_H):
        acc = acc + jnp.dot(
            a_ref[0, h], wo_ref[h * _Dh:(h + 1) * _Dh, :],
            preferred_element_type=jnp.float32)
    o_ref[0] = acc


def _out_proj(att, wo):
    return pl.pallas_call(
        _proj_kernel,
        grid=(_B,),
        in_specs=[
            pl.BlockSpec((1, _H, _NM, _Dh), lambda b: (b, 0, 0, 0)),
            pl.BlockSpec((_C, _C), lambda b: (0, 0)),
        ],
        out_specs=pl.BlockSpec((1, _NM, _C), lambda b: (b, 0, 0)),
        out_shape=jax.ShapeDtypeStruct((_B, _NM, _C), jnp.float32),
    )(att, wo)


_sc_mesh = plsc.VectorSubcoreMesh(
    core_axis_name="c", subcore_axis_name="s", num_cores=_NC)


@functools.partial(
    pl.kernel, mesh=_sc_mesh,
    out_type=jax.ShapeDtypeStruct((_B * _NM, _C), jnp.float32),
    scratch_types=[
        pltpu.VMEM((_MCH,), jnp.int32),
        pltpu.VMEM((_MCH,), jnp.int32),
        pltpu.VMEM((_MCH, _C), jnp.float32),
        pltpu.VMEM((_MCH, _C), jnp.float32),
        pltpu.SemaphoreType.DMA,
        pltpu.SemaphoreType.DMA,
    ])
def _sc_merge(x_hbm, i1_hbm, i2_hbm, out_hbm, i1_v, i2_v, r1_v, r2_v, s1, s2):
    wid = lax.axis_index("s") * _NC + lax.axis_index("c")
    base = wid * _MPW
    for c in range(_MPW // _MCH):
        co = base + c * _MCH
        pltpu.sync_copy(i1_hbm.at[pl.ds(co, _MCH)], i1_v)
        pltpu.sync_copy(i2_hbm.at[pl.ds(co, _MCH)], i2_v)
        cp1 = pltpu.async_copy(x_hbm.at[i1_v], r1_v, s1)
        cp2 = pltpu.async_copy(x_hbm.at[i2_v], r2_v, s2)
        cp1.wait()
        cp2.wait()

        def row_body(i, _):
            def grp_body(j, _):
                sl = pl.ds(j * 16, 16)
                r1_v[i, sl] = (r1_v[i, sl] + r2_v[i, sl]) * 0.5
                return 0
            return lax.fori_loop(0, _C // 16, grp_body, 0)

        lax.fori_loop(0, _MCH, row_body, 0)
        pltpu.sync_copy(r1_v, out_hbm.at[pl.ds(co, _MCH)])


@functools.partial(
    pl.kernel, mesh=_sc_mesh,
    out_type=jax.ShapeDtypeStruct((_B * _N, _C), jnp.float32),
    scratch_types=[
        pltpu.VMEM((_UPW,), jnp.int32),
        pltpu.VMEM((_UPW, _C), jnp.float32),
        pltpu.SemaphoreType.DMA,
    ])
def _sc_unmerge(tbl_hbm, src_hbm, out_hbm, idx_v, rows_v, sem):
    wid = lax.axis_index("s") * _NC + lax.axis_index("c")
    base = wid * _UPW
    pltpu.sync_copy(src_hbm.at[pl.ds(base, _UPW)], idx_v)
    pltpu.async_copy(tbl_hbm.at[idx_v], rows_v, sem).wait()
    pltpu.sync_copy(rows_v, out_hbm.at[pl.ds(base, _UPW)])


def _select_pairs(best_s, best_b):
    bi = jnp.arange(_B, dtype=jnp.int32)[:, None]
    order = jnp.argsort(-best_s, axis=1).astype(jnp.int32)
    bb_ord = jnp.take_along_axis(best_b, order, axis=1)
    pos = jnp.arange(_HALF, dtype=jnp.int32)[None, :]
    firstpos = jnp.full((_B, _HALF), _HALF, dtype=jnp.int32)
    firstpos = firstpos.at[bi, bb_ord].min(
        jnp.broadcast_to(pos, (_B, _HALF)))
    take0 = jnp.take_along_axis(firstpos, bb_ord, axis=1) == pos
    c0 = jnp.cumsum(take0.astype(jnp.int32), axis=1) - take0.astype(jnp.int32)
    take = jnp.logical_and(take0, c0 < _R)
    cnt = jnp.sum(take.astype(jnp.int32), axis=1)
    sel = jnp.argsort(jnp.logical_not(take).astype(jnp.int32),
                      axis=1).astype(jnp.int32)[:, :_R]
    ca = jnp.take_along_axis(order, sel, axis=1)
    slots = jnp.arange(_R, dtype=jnp.int32)[None, :]
    ca = jnp.where(slots < cnt[:, None], ca, ca[:, :1])
    cb = jnp.take_along_axis(best_b, ca, axis=1)
    ga = ca * 2
    gb = cb * 2 + 1
    rm = jnp.zeros((_B, _N), dtype=bool).at[bi, gb].set(True)
    keep = jnp.argsort(rm.astype(jnp.int32), axis=1).astype(jnp.int32)[:, :_NM]
    return ga, gb, keep, rm


def _build_s_mat():
    s = np.zeros((_HP * _Dh, _HP * _Dh), dtype=np.float32)
    for h in range(_HP):
        o = h * _Dh
        for i in range(_Dh // 2):
            s[o + 2 * i + 1, o + 2 * i] = -1.0
            s[o + 2 * i, o + 2 * i + 1] = 1.0
    return jnp.asarray(s)


@jax.jit
def _run(x, freqs_cis, wq, wk, wv, wo):
    bi = jnp.arange(_B, dtype=jnp.int32)[:, None]

    x_even = x[:, 0::2, :]
    x_odd = x[:, 1::2, :]
    best_s, best_b = _match(x_even, x_odd, wk)
    ga, gb, keep, rm = _select_pairs(best_s, best_b)

    pmerge = jnp.broadcast_to(
        jnp.arange(_N, dtype=jnp.int32)[None, :], (_B, _N))
    pmerge = pmerge.at[bi, ga].set(gb)
    idx2 = jnp.take_along_axis(pmerge, keep, axis=1)
    i1f = (keep + bi * _N).reshape(-1)
    i2f = (idx2 + bi * _N).reshape(-1)
    x_m = _sc_merge(x.reshape(_B * _N, _C), i1f, i2f).reshape(_B, _NM, _C)

    cos = freqs_cis[:_NM, :, 0]
    sin = freqs_cis[:_NM, :, 1]
    cos_i = jnp.tile(jnp.repeat(cos, 2, axis=1), (1, _HP))
    sin_i = jnp.tile(jnp.repeat(sin, 2, axis=1), (1, _HP))
    s_mat = _build_s_mat()

    q, k, v = _qkv(x_m, wq, wk, wv, cos_i, sin_i, s_mat)
    att = _attention(q, k, v)
    out_m = _out_proj(att, wo)

    rows = jnp.zeros((_B, _N), dtype=jnp.int32).at[bi, keep].set(
        jnp.broadcast_to(jnp.arange(_NM, dtype=jnp.int32)[None, :],
                         (_B, _NM)))
    in_keep = jnp.zeros((_B, _N), dtype=bool).at[bi, keep].set(True)
    pb = jnp.broadcast_to(
        jnp.arange(_N, dtype=jnp.int32)[None, :], (_B, _N))
    pb = pb.at[bi, gb].set(ga)
    src = jnp.where(rm, jnp.take_along_axis(rows, pb, axis=1), rows)
    valid = jnp.logical_or(
        in_keep,
        jnp.logical_and(rm, jnp.take_along_axis(in_keep, pb, axis=1)))
    srcf = jnp.where(valid, src + bi * _NM, _ZROW).reshape(-1)
    tbl = jnp.concatenate(
        [out_m.reshape(_B * _NM, _C),
         jnp.zeros((_PAD, _C), jnp.float32)], axis=0)
    out = _sc_unmerge(tbl, srcf).reshape(_B, _N, _C)
    return out, k, v


def kernel(x, freqs_cis, Wq, Wk, Wv, Wo):
    return _run(x, freqs_cis, Wq, Wk, Wv, Wo)


# speedup vs baseline: 1.3827x; 1.3788x over previous
import functools

import jax
import jax.numpy as jnp
import numpy as np
from jax import lax
from jax.experimental import pallas as pl
from jax.experimental.pallas import tpu as pltpu
from jax.experimental.pallas import tpu_sc as plsc

_B, _N, _C = 2, 2048, 768
_H = 12
_Dh = _C // _H
_R = 256
_HALF = _N // 2
_NM = _N - _R

_NC = 2
_NS = 16
_NW = _NC * _NS
_MPW = (_B * _NM) // _NW
_MCH = _MPW // 2
_UPW = (_B * _N) // _NW
_PAD = 8
_ZROW = _B * _NM


def _scores_kernel(xe_ref, xo_ref, wk_ref, bs_ref, bb_ref):
    a = jnp.dot(xe_ref[0], wk_ref[...], preferred_element_type=jnp.float32)
    b = jnp.dot(xo_ref[0], wk_ref[...], preferred_element_type=jnp.float32)
    an = a / jnp.maximum(
        jnp.sqrt(jnp.sum(a * a, axis=1, keepdims=True)), 1e-12)
    bn = b / jnp.maximum(
        jnp.sqrt(jnp.sum(b * b, axis=1, keepdims=True)), 1e-12)
    scores = jax.lax.dot_general(
        an, bn, (((1,), (1,)), ((), ())),
        preferred_element_type=jnp.float32)
    ms = jnp.max(scores, axis=1, keepdims=True)
    cols = jax.lax.broadcasted_iota(jnp.int32, scores.shape, 1)
    cand = jnp.where(scores == ms, cols, jnp.int32(_HALF))
    bs_ref[0, 0, :] = ms[:, 0]
    bb_ref[0, 0, :] = jnp.min(cand, axis=1)


_SCH = 128


def _select_kernel(bs_ref, bb_ref, dste_ref, dsto_ref, srco_ref, pe_ref):
    bs = bs_ref[0, 0, :]
    bb = bb_ref[0, 0, :]
    bs_r = bs[None, :]
    bb_r = bb[None, :]
    col_i = jax.lax.broadcasted_iota(jnp.int32, (_SCH, _HALF), 1)
    nch = _HALF // _SCH

    ws_parts, wj_parts = [], []
    for c in range(nch):
        v_col = jax.lax.broadcasted_iota(
            jnp.int32, (_SCH, 1), 0) + jnp.int32(c * _SCH)
        m = bb_r == v_col
        ws_c = jnp.max(jnp.where(m, bs_r, jnp.float32(-2.0)), axis=1)
        wj_cand = jnp.where(
            jnp.logical_and(m, bs_r == ws_c[:, None]),
            col_i, jnp.int32(_HALF))
        ws_parts.append(ws_c)
        wj_parts.append(jnp.min(wj_cand, axis=1))
    ws = jnp.concatenate(ws_parts)
    wj = jnp.concatenate(wj_parts)
    valid = ws > jnp.float32(-1.5)

    ws_r = ws[None, :]
    wj_r = wj[None, :]
    valid_r = valid[None, :]
    rank_parts = []
    for c in range(nch):
        sl = slice(c * _SCH, (c + 1) * _SCH)
        ws_c = ws[sl][:, None]
        wj_c = wj[sl][:, None]
        better = jnp.logical_and(
            valid_r,
            jnp.logical_or(
                ws_r > ws_c,
                jnp.logical_and(ws_r == ws_c, wj_r < wj_c)))
        rank_parts.append(jnp.sum(better.astype(jnp.int32), axis=1))
    rank = jnp.concatenate(rank_parts)
    take = jnp.logical_and(valid, rank < _R)

    take8 = take.astype(jnp.float32).reshape(8, _HALF // 8)
    w = _HALF // 8
    ci = jax.lax.broadcasted_iota(jnp.int32, (w, w), 0)
    cj = jax.lax.broadcasted_iota(jnp.int32, (w, w), 1)
    u_incl = (ci <= cj).astype(jnp.float32)
    incl8 = jnp.dot(take8, u_incl, preferred_element_type=jnp.float32)
    rowsum = incl8[:, w - 1:w]
    ri = jax.lax.broadcasted_iota(jnp.int32, (8, 8), 0)
    rj = jax.lax.broadcasted_iota(jnp.int32, (8, 8), 1)
    s_strict = (rj < ri).astype(jnp.float32)
    rowoff = jnp.dot(s_strict, rowsum,
                     preferred_element_type=jnp.float32)
    tk = (incl8 + rowoff).reshape(_HALF) - take.astype(jnp.float32)

    v_iota_f = jax.lax.broadcasted_iota(
        jnp.int32, (_HALF,), 0).astype(jnp.float32)
    pos_even = 2.0 * v_iota_f - tk
    pos_odd = 2.0 * v_iota_f + 1.0 - tk

    wj_f = wj.astype(jnp.float32)
    take_f = take.astype(jnp.float32)
    wj_at_parts, tk_at_parts, posj_parts = [], [], []
    for c in range(nch):
        sl = slice(c * _SCH, (c + 1) * _SCH)
        a_uv = (bb[sl][:, None] == col_i).astype(jnp.float32)
        wj_at_parts.append(jnp.dot(a_uv, wj_f,
                                   preferred_element_type=jnp.float32))
        tk_at_parts.append(jnp.dot(a_uv, take_f,
                                   preferred_element_type=jnp.float32))
        o3 = (wj[sl][:, None] == col_i).astype(jnp.float32)
        posj_parts.append(jnp.dot(o3, pos_even,
                                  preferred_element_type=jnp.float32))
    wj_at_u = jnp.concatenate(wj_at_parts)
    tk_at_u = jnp.concatenate(tk_at_parts)
    posj = jnp.concatenate(posj_parts)

    u_iota_f = v_iota_f
    merged_a = jnp.logical_and(tk_at_u > 0.5, wj_at_u == u_iota_f)

    nm_f = jnp.float32(_NM)
    sent = jnp.int32(_NM)
    pos_even_ok = pos_even < nm_f
    pos_odd_ok = pos_odd < nm_f
    dst_e = jnp.where(pos_even_ok, pos_even.astype(jnp.int32), sent)
    keep_odd = jnp.logical_not(take)
    dst_o = jnp.where(jnp.logical_and(keep_odd, pos_odd_ok),
                      pos_odd.astype(jnp.int32), sent)
    src_o = jnp.where(
        take,
        jnp.where(posj < nm_f, posj.astype(jnp.int32), sent),
        jnp.where(pos_odd_ok, pos_odd.astype(jnp.int32), sent))
    v_iota_i = jax.lax.broadcasted_iota(jnp.int32, (_HALF,), 0)
    pe = jnp.where(merged_a, 2 * bb + 1, 2 * v_iota_i)

    dste_ref[0, 0, :] = dst_e
    dsto_ref[0, 0, :] = dst_o
    srco_ref[0, 0, :] = src_o
    pe_ref[0, 0, :] = pe


def _match(x_even, x_odd, wk):
    bs, bb = pl.pallas_call(
        _scores_kernel,
        grid=(_B,),
        in_specs=[
            pl.BlockSpec((1, _HALF, _C), lambda i: (i, 0, 0)),
            pl.BlockSpec((1, _HALF, _C), lambda i: (i, 0, 0)),
            pl.BlockSpec((_C, _C), lambda i: (0, 0)),
        ],
        out_specs=[
            pl.BlockSpec((1, 1, _HALF), lambda i: (i, 0, 0)),
            pl.BlockSpec((1, 1, _HALF), lambda i: (i, 0, 0)),
        ],
        out_shape=[
            jax.ShapeDtypeStruct((_B, 1, _HALF), jnp.float32),
            jax.ShapeDtypeStruct((_B, 1, _HALF), jnp.int32),
        ],
    )(x_even, x_odd, wk)
    outs = pl.pallas_call(
        _select_kernel,
        grid=(_B,),
        in_specs=[
            pl.BlockSpec((1, 1, _HALF), lambda i: (i, 0, 0)),
            pl.BlockSpec((1, 1, _HALF), lambda i: (i, 0, 0)),
        ],
        out_specs=[
            pl.BlockSpec((1, 1, _HALF), lambda i: (i, 0, 0)),
            pl.BlockSpec((1, 1, _HALF), lambda i: (i, 0, 0)),
            pl.BlockSpec((1, 1, _HALF), lambda i: (i, 0, 0)),
            pl.BlockSpec((1, 1, _HALF), lambda i: (i, 0, 0)),
        ],
        out_shape=[
            jax.ShapeDtypeStruct((_B, 1, _HALF), jnp.int32),
            jax.ShapeDtypeStruct((_B, 1, _HALF), jnp.int32),
            jax.ShapeDtypeStruct((_B, 1, _HALF), jnp.int32),
            jax.ShapeDtypeStruct((_B, 1, _HALF), jnp.int32),
        ],
    )(bs, bb)
    return tuple(o[:, 0, :] for o in outs)


_HP = 2
_HB = _H // _HP


def _qkv_kernel(xm_ref, wq_ref, wk_ref, wv_ref, cos_ref, sin_ref, s_ref,
                q_ref, k_ref, v_ref):
    x = xm_ref[...]
    cos = cos_ref[...]
    sin = sin_ref[...]
    s_mat = s_ref[...]

    q = jnp.dot(x, wq_ref[...], preferred_element_type=jnp.float32)
    q_rot = jnp.dot(q, s_mat, preferred_element_type=jnp.float32)
    qr = q * cos + q_rot * sin
    q_ref[0, 0] = qr[:, :_Dh]
    q_ref[0, 1] = qr[:, _Dh:]

    k = jnp.dot(x, wk_ref[...], preferred_element_type=jnp.float32)
    k_rot = jnp.dot(k, s_mat, preferred_element_type=jnp.float32)
    kr = k * cos + k_rot * sin
    k_ref[0, 0] = kr[:, :_Dh]
    k_ref[0, 1] = kr[:, _Dh:]

    v = jnp.dot(x, wv_ref[...], preferred_element_type=jnp.float32)
    v_ref[0, 0] = v[:, :_Dh]
    v_ref[0, 1] = v[:, _Dh:]


def _qkv(x_m, wq, wk, wv, cos_i, sin_i, s_mat):
    wcols = _HP * _Dh
    return pl.pallas_call(
        _qkv_kernel,
        grid=(_B, _HB),
        in_specs=[
            pl.BlockSpec((_NM, _C), lambda b, h: (b, 0)),
            pl.BlockSpec((_C, wcols), lambda b, h: (0, h)),
            pl.BlockSpec((_C, wcols), lambda b, h: (0, h)),
            pl.BlockSpec((_C, wcols), lambda b, h: (0, h)),
            pl.BlockSpec((_NM, wcols), lambda b, h: (0, 0)),
            pl.BlockSpec((_NM, wcols), lambda b, h: (0, 0)),
            pl.BlockSpec((wcols, wcols), lambda b, h: (0, 0)),
        ],
        out_specs=[
            pl.BlockSpec((1, _HP, _NM, _Dh), lambda b, h: (b, h, 0, 0)),
            pl.BlockSpec((1, _HP, _NM, _Dh), lambda b, h: (b, h, 0, 0)),
            pl.BlockSpec((1, _HP, _NM, _Dh), lambda b, h: (b, h, 0, 0)),
        ],
        out_shape=[
            jax.ShapeDtypeStruct((_B, _H, _NM, _Dh), jnp.float32),
            jax.ShapeDtypeStruct((_B, _H, _NM, _Dh), jnp.float32),
            jax.ShapeDtypeStruct((_B, _H, _NM, _Dh), jnp.float32),
        ],
    )(x_m, wq, wk, wv, cos_i, sin_i, s_mat)


def _attn_kernel(q_ref, k_ref, v_ref, o_ref):
    q = q_ref[0, 0].astype(jnp.bfloat16)
    k = k_ref[0, 0].astype(jnp.bfloat16)
    v = v_ref[0, 0].astype(jnp.bfloat16)
    s = jax.lax.dot_general(
        q, k, (((1,), (1,)), ((), ())),
        preferred_element_type=jnp.float32)
    s = s * jnp.float32(1.0 / np.sqrt(_Dh))
    rows = jax.lax.broadcasted_iota(jnp.int32, s.shape, 0)
    cols = jax.lax.broadcasted_iota(jnp.int32, s.shape, 1)
    s = jnp.where(rows >= cols, s, jnp.float32(-1e9))
    m = jnp.max(s, axis=1, keepdims=True)
    e = jnp.exp(s - m)
    p = (e / jnp.sum(e, axis=1, keepdims=True)).astype(jnp.bfloat16)
    o_ref[0, 0] = jnp.dot(p, v, preferred_element_type=jnp.float32)


def _attention(q, k, v):
    return pl.pallas_call(
        _attn_kernel,
        grid=(_B, _H),
        in_specs=[
            pl.BlockSpec((1, 1, _NM, _Dh), lambda b, h: (b, h, 0, 0)),
            pl.BlockSpec((1, 1, _NM, _Dh), lambda b, h: (b, h, 0, 0)),
            pl.BlockSpec((1, 1, _NM, _Dh), lambda b, h: (b, h, 0, 0)),
        ],
        out_specs=pl.BlockSpec((1, 1, _NM, _Dh), lambda b, h: (b, h, 0, 0)),
        out_shape=jax.ShapeDtypeStruct((_B, _H, _NM, _Dh), jnp.float32),
    )(q, k, v)


def _proj_kernel(a_ref, wo_ref, o_ref):
    acc = jnp.zeros((_NM, _C), dtype=jnp.float32)
    for h in range(_H):
        acc = acc + jnp.dot(
            a_ref[0, h], wo_ref[h * _Dh:(h + 1) * _Dh, :],
            preferred_element_type=jnp.float32)
    o_ref[0] = acc


def _out_proj(att, wo):
    return pl.pallas_call(
        _proj_kernel,
        grid=(_B,),
        in_specs=[
            pl.BlockSpec((1, _H, _NM, _Dh), lambda b: (b, 0, 0, 0)),
            pl.BlockSpec((_C, _C), lambda b: (0, 0)),
        ],
        out_specs=pl.BlockSpec((1, _NM, _C), lambda b: (b, 0, 0)),
        out_shape=jax.ShapeDtypeStruct((_B, _NM, _C), jnp.float32),
    )(att, wo)


_TPW = (_B * _N) // _NW
_TCH = _TPW // 2


@functools.lru_cache(maxsize=1)
def _sc_kernels():
    mesh = plsc.VectorSubcoreMesh(
        core_axis_name="c", subcore_axis_name="s", num_cores=_NC)

    @functools.partial(
        pl.kernel, mesh=mesh,
        out_type=jax.ShapeDtypeStruct((_B * _NM + _PAD, _C), jnp.float32),
        scratch_types=[
            pltpu.VMEM((_TCH,), jnp.int32),
            pltpu.VMEM((_TCH,), jnp.int32),
            pltpu.VMEM((_TCH, _C), jnp.float32),
            pltpu.VMEM((_TCH, _C), jnp.float32),
            pltpu.SemaphoreType.DMA,
            pltpu.SemaphoreType.DMA,
        ])
    def sc_merge(x_hbm, pf_hbm, dst_hbm, out_hbm, pf_v, dst_v, xr_v, pr_v,
                 s1, s2):
        wid = lax.axis_index("s") * _NC + lax.axis_index("c")
        base = wid * _TPW
        for c in range(_TPW // _TCH):
            co = base + c * _TCH
            pltpu.sync_copy(pf_hbm.at[pl.ds(co, _TCH)], pf_v)
            pltpu.sync_copy(dst_hbm.at[pl.ds(co, _TCH)], dst_v)
            cp1 = pltpu.async_copy(x_hbm.at[pl.ds(co, _TCH)], xr_v, s1)
            cp2 = pltpu.async_copy(x_hbm.at[pf_v], pr_v, s2)
            cp1.wait()
            cp2.wait()

            def row_body(i, _):
                def grp_body(j, _):
                    sl = pl.ds(j * 16, 16)
                    xr_v[i, sl] = (xr_v[i, sl] + pr_v[i, sl]) * 0.5
                    return 0
                return lax.fori_loop(0, _C // 16, grp_body, 0)

            lax.fori_loop(0, _TCH, row_body, 0)
            pltpu.async_copy(xr_v, out_hbm.at[dst_v], s1).wait()

    @functools.partial(
        pl.kernel, mesh=mesh,
        out_type=jax.ShapeDtypeStruct((_B * _N, _C), jnp.float32),
        scratch_types=[
            pltpu.VMEM((_UPW,), jnp.int32),
            pltpu.VMEM((_UPW, _C), jnp.float32),
            pltpu.SemaphoreType.DMA,
        ])
    def sc_unmerge(tbl_hbm, src_hbm, out_hbm, idx_v, rows_v, sem):
        wid = lax.axis_index("s") * _NC + lax.axis_index("c")
        base = wid * _UPW
        pltpu.sync_copy(src_hbm.at[pl.ds(base, _UPW)], idx_v)
        pltpu.async_copy(tbl_hbm.at[idx_v], rows_v, sem).wait()
        pltpu.sync_copy(rows_v, out_hbm.at[pl.ds(base, _UPW)])

    return sc_merge, sc_unmerge


def _sc_merge(xf, pf, dstf):
    return _sc_kernels()[0](xf, pf, dstf)


def _sc_unmerge(tbl, srcf):
    return _sc_kernels()[1](tbl, srcf)


def _build_s_mat():
    s = np.zeros((_HP * _Dh, _HP * _Dh), dtype=np.float32)
    for h in range(_HP):
        o = h * _Dh
        for i in range(_Dh // 2):
            s[o + 2 * i + 1, o + 2 * i] = -1.0
            s[o + 2 * i, o + 2 * i + 1] = 1.0
    return jnp.asarray(s)


@jax.jit
def _run(x, freqs_cis, wq, wk, wv, wo):
    bi = jnp.arange(_B, dtype=jnp.int32)[:, None]

    x_even = x[:, 0::2, :]
    x_odd = x[:, 1::2, :]
    dst_e, dst_o, src_o, pe = _match(x_even, x_odd, wk)

    odd_ids = jnp.broadcast_to(
        (2 * jnp.arange(_HALF, dtype=jnp.int32) + 1)[None, :], (_B, _HALF))
    dst = jnp.stack([dst_e, dst_o], axis=-1).reshape(_B, _N)
    src = jnp.stack([dst_e, src_o], axis=-1).reshape(_B, _N)
    partner = jnp.stack([pe, odd_ids], axis=-1).reshape(_B, _N)

    trash = jnp.int32(_ZROW)
    dstf = jnp.where(dst < _NM, dst + bi * _NM, trash).reshape(-1)
    srcf = jnp.where(src < _NM, src + bi * _NM, trash).reshape(-1)
    pf = (partner + bi * _N).reshape(-1)

    x_mp = _sc_merge(x.reshape(_B * _N, _C), pf, dstf)

    cos = freqs_cis[:_NM, :, 0]
    sin = freqs_cis[:_NM, :, 1]
    cos_i = jnp.tile(jnp.repeat(cos, 2, axis=1), (1, _HP))
    sin_i = jnp.tile(jnp.repeat(sin, 2, axis=1), (1, _HP))
    s_mat = _build_s_mat()

    q, k, v = _qkv(x_mp, wq, wk, wv, cos_i, sin_i, s_mat)
    att = _attention(q, k, v)
    out_m = _out_proj(att, wo)

    tbl = jnp.concatenate(
        [out_m.reshape(_B * _NM, _C),
         jnp.zeros((_PAD, _C), jnp.float32)], axis=0)
    out = _sc_unmerge(tbl, srcf).reshape(_B, _N, _C)
    return out, k, v


def kernel(x, freqs_cis, Wq, Wk, Wv, Wo):
    return _run(x, freqs_cis, Wq, Wk, Wv, Wo)


# speedup vs baseline: 1.5287x; 1.1056x over previous
import functools

import jax
import jax.numpy as jnp
import numpy as np
from jax import lax
from jax.experimental import pallas as pl
from jax.experimental.pallas import tpu as pltpu
from jax.experimental.pallas import tpu_sc as plsc

_B, _N, _C = 2, 2048, 768
_H = 12
_Dh = _C // _H
_R = 256
_HALF = _N // 2
_NM = _N - _R

_NC = 2
_NS = 16
_NW = _NC * _NS
_MPW = (_B * _NM) // _NW
_MCH = _MPW // 2
_UPW = (_B * _N) // _NW
_PAD = 8
_ZROW = _B * _NM


def _scores_kernel(x2_ref, wk_ref, bs_ref, bb_ref):
    x2 = x2_ref[0]
    a = jnp.dot(x2[:, :_C], wk_ref[...], preferred_element_type=jnp.float32)
    b = jnp.dot(x2[:, _C:], wk_ref[...], preferred_element_type=jnp.float32)
    an = a / jnp.maximum(
        jnp.sqrt(jnp.sum(a * a, axis=1, keepdims=True)), 1e-12)
    bn = b / jnp.maximum(
        jnp.sqrt(jnp.sum(b * b, axis=1, keepdims=True)), 1e-12)
    scores = jax.lax.dot_general(
        an, bn, (((1,), (1,)), ((), ())),
        preferred_element_type=jnp.float32)
    ms = jnp.max(scores, axis=1, keepdims=True)
    cols = jax.lax.broadcasted_iota(jnp.int32, scores.shape, 1)
    cand = jnp.where(scores == ms, cols, jnp.int32(_HALF))
    bs_ref[0, 0, :] = ms[:, 0]
    bb_ref[0, 0, :] = jnp.min(cand, axis=1)


_SCH = 128


def _select_kernel(bs_ref, bb_ref, dste_ref, dsto_ref, srco_ref, pe_ref):
    bs = bs_ref[0, 0, :]
    bb = bb_ref[0, 0, :]
    bs_r = bs[None, :]
    bb_r = bb[None, :]
    col_i = jax.lax.broadcasted_iota(jnp.int32, (_SCH, _HALF), 1)
    nch = _HALF // _SCH

    ws_parts, wj_parts = [], []
    for c in range(nch):
        v_col = jax.lax.broadcasted_iota(
            jnp.int32, (_SCH, 1), 0) + jnp.int32(c * _SCH)
        m = bb_r == v_col
        ws_c = jnp.max(jnp.where(m, bs_r, jnp.float32(-2.0)), axis=1)
        wj_cand = jnp.where(
            jnp.logical_and(m, bs_r == ws_c[:, None]),
            col_i, jnp.int32(_HALF))
        ws_parts.append(ws_c)
        wj_parts.append(jnp.min(wj_cand, axis=1))
    ws = jnp.concatenate(ws_parts)
    wj = jnp.concatenate(wj_parts)
    valid = ws > jnp.float32(-1.5)

    ws_r = ws[None, :]
    wj_r = wj[None, :]
    valid_r = valid[None, :]
    rank_parts = []
    for c in range(nch):
        sl = slice(c * _SCH, (c + 1) * _SCH)
        ws_c = ws[sl][:, None]
        wj_c = wj[sl][:, None]
        better = jnp.logical_and(
            valid_r,
            jnp.logical_or(
                ws_r > ws_c,
                jnp.logical_and(ws_r == ws_c, wj_r < wj_c)))
        rank_parts.append(jnp.sum(better.astype(jnp.int32), axis=1))
    rank = jnp.concatenate(rank_parts)
    take = jnp.logical_and(valid, rank < _R)

    take8 = take.astype(jnp.float32).reshape(8, _HALF // 8)
    w = _HALF // 8
    ci = jax.lax.broadcasted_iota(jnp.int32, (w, w), 0)
    cj = jax.lax.broadcasted_iota(jnp.int32, (w, w), 1)
    u_incl = (ci <= cj).astype(jnp.float32)
    incl8 = jnp.dot(take8, u_incl, preferred_element_type=jnp.float32)
    rowsum = incl8[:, w - 1:w]
    ri = jax.lax.broadcasted_iota(jnp.int32, (8, 8), 0)
    rj = jax.lax.broadcasted_iota(jnp.int32, (8, 8), 1)
    s_strict = (rj < ri).astype(jnp.float32)
    rowoff = jnp.dot(s_strict, rowsum,
                     preferred_element_type=jnp.float32)
    tk = (incl8 + rowoff).reshape(_HALF) - take.astype(jnp.float32)

    v_iota_f = jax.lax.broadcasted_iota(
        jnp.int32, (_HALF,), 0).astype(jnp.float32)
    pos_even = 2.0 * v_iota_f - tk
    pos_odd = 2.0 * v_iota_f + 1.0 - tk

    wj_f = wj.astype(jnp.float32)
    take_f = take.astype(jnp.float32)
    wj_at_parts, tk_at_parts, posj_parts = [], [], []
    for c in range(nch):
        sl = slice(c * _SCH, (c + 1) * _SCH)
        a_uv = (bb[sl][:, None] == col_i).astype(jnp.float32)
        wj_at_parts.append(jnp.dot(a_uv, wj_f,
                                   preferred_element_type=jnp.float32))
        tk_at_parts.append(jnp.dot(a_uv, take_f,
                                   preferred_element_type=jnp.float32))
        o3 = (wj[sl][:, None] == col_i).astype(jnp.float32)
        posj_parts.append(jnp.dot(o3, pos_even,
                                  preferred_element_type=jnp.float32))
    wj_at_u = jnp.concatenate(wj_at_parts)
    tk_at_u = jnp.concatenate(tk_at_parts)
    posj = jnp.concatenate(posj_parts)

    u_iota_f = v_iota_f
    merged_a = jnp.logical_and(tk_at_u > 0.5, wj_at_u == u_iota_f)

    nm_f = jnp.float32(_NM)
    sent = jnp.int32(_NM)
    pos_even_ok = pos_even < nm_f
    pos_odd_ok = pos_odd < nm_f
    dst_e = jnp.where(pos_even_ok, pos_even.astype(jnp.int32), sent)
    keep_odd = jnp.logical_not(take)
    dst_o = jnp.where(jnp.logical_and(keep_odd, pos_odd_ok),
                      pos_odd.astype(jnp.int32), sent)
    src_o = jnp.where(
        take,
        jnp.where(posj < nm_f, posj.astype(jnp.int32), sent),
        jnp.where(pos_odd_ok, pos_odd.astype(jnp.int32), sent))
    v_iota_i = jax.lax.broadcasted_iota(jnp.int32, (_HALF,), 0)
    pe = jnp.where(merged_a, 2 * bb + 1, 2 * v_iota_i)

    dste_ref[0, 0, :] = dst_e
    dsto_ref[0, 0, :] = dst_o
    srco_ref[0, 0, :] = src_o
    pe_ref[0, 0, :] = pe


def _match(x2, wk):
    bs, bb = pl.pallas_call(
        _scores_kernel,
        grid=(_B,),
        in_specs=[
            pl.BlockSpec((1, _HALF, 2 * _C), lambda i: (i, 0, 0)),
            pl.BlockSpec((_C, _C), lambda i: (0, 0)),
        ],
        out_specs=[
            pl.BlockSpec((1, 1, _HALF), lambda i: (i, 0, 0)),
            pl.BlockSpec((1, 1, _HALF), lambda i: (i, 0, 0)),
        ],
        out_shape=[
            jax.ShapeDtypeStruct((_B, 1, _HALF), jnp.float32),
            jax.ShapeDtypeStruct((_B, 1, _HALF), jnp.int32),
        ],
    )(x2, wk)
    outs = pl.pallas_call(
        _select_kernel,
        grid=(_B,),
        in_specs=[
            pl.BlockSpec((1, 1, _HALF), lambda i: (i, 0, 0)),
            pl.BlockSpec((1, 1, _HALF), lambda i: (i, 0, 0)),
        ],
        out_specs=[
            pl.BlockSpec((1, 1, _HALF), lambda i: (i, 0, 0)),
            pl.BlockSpec((1, 1, _HALF), lambda i: (i, 0, 0)),
            pl.BlockSpec((1, 1, _HALF), lambda i: (i, 0, 0)),
            pl.BlockSpec((1, 1, _HALF), lambda i: (i, 0, 0)),
        ],
        out_shape=[
            jax.ShapeDtypeStruct((_B, 1, _HALF), jnp.int32),
            jax.ShapeDtypeStruct((_B, 1, _HALF), jnp.int32),
            jax.ShapeDtypeStruct((_B, 1, _HALF), jnp.int32),
            jax.ShapeDtypeStruct((_B, 1, _HALF), jnp.int32),
        ],
    )(bs, bb)
    return tuple(o[:, 0, :] for o in outs)


_HP = 2
_HB = _H // _HP


def _qkv_kernel(xm_ref, wq_ref, wk_ref, wv_ref, cos_ref, sin_ref, e_ref,
                s_ref, q_ref, k_ref, v_ref):
    x = xm_ref[...]
    e_mat = e_ref[...]
    cos = jnp.dot(cos_ref[...], e_mat, preferred_element_type=jnp.float32)
    sin = jnp.dot(sin_ref[...], e_mat, preferred_element_type=jnp.float32)
    s_mat = s_ref[...]

    q = jnp.dot(x, wq_ref[...], preferred_element_type=jnp.float32)
    q_rot = jnp.dot(q, s_mat, preferred_element_type=jnp.float32)
    qr = q * cos + q_rot * sin
    q_ref[0, 0] = qr[:, :_Dh]
    q_ref[0, 1] = qr[:, _Dh:]

    k = jnp.dot(x, wk_ref[...], preferred_element_type=jnp.float32)
    k_rot = jnp.dot(k, s_mat, preferred_element_type=jnp.float32)
    kr = k * cos + k_rot * sin
    k_ref[0, 0] = kr[:, :_Dh]
    k_ref[0, 1] = kr[:, _Dh:]

    v = jnp.dot(x, wv_ref[...], preferred_element_type=jnp.float32)
    v_ref[0, 0] = v[:, :_Dh]
    v_ref[0, 1] = v[:, _Dh:]


def _qkv(x_m, wq, wk, wv, cos_i, sin_i, e_mat, s_mat):
    wcols = _HP * _Dh
    return pl.pallas_call(
        _qkv_kernel,
        grid=(_B, _HB),
        in_specs=[
            pl.BlockSpec((_NM, _C), lambda b, h: (b, 0)),
            pl.BlockSpec((_C, wcols), lambda b, h: (0, h)),
            pl.BlockSpec((_C, wcols), lambda b, h: (0, h)),
            pl.BlockSpec((_C, wcols), lambda b, h: (0, h)),
            pl.BlockSpec((_NM, _Dh // 2), lambda b, h: (0, 0)),
            pl.BlockSpec((_NM, _Dh // 2), lambda b, h: (0, 0)),
            pl.BlockSpec((_Dh // 2, wcols), lambda b, h: (0, 0)),
            pl.BlockSpec((wcols, wcols), lambda b, h: (0, 0)),
        ],
        out_specs=[
            pl.BlockSpec((1, _HP, _NM, _Dh), lambda b, h: (b, h, 0, 0)),
            pl.BlockSpec((1, _HP, _NM, _Dh), lambda b, h: (b, h, 0, 0)),
            pl.BlockSpec((1, _HP, _NM, _Dh), lambda b, h: (b, h, 0, 0)),
        ],
        out_shape=[
            jax.ShapeDtypeStruct((_B, _H, _NM, _Dh), jnp.float32),
            jax.ShapeDtypeStruct((_B, _H, _NM, _Dh), jnp.float32),
            jax.ShapeDtypeStruct((_B, _H, _NM, _Dh), jnp.float32),
        ],
    )(x_m, wq, wk, wv, cos_i, sin_i, e_mat, s_mat)


def _attn_kernel(q_ref, k_ref, v_ref, o_ref):
    q = (q_ref[0, 0] * jnp.float32(1.0 / np.sqrt(_Dh))).astype(jnp.bfloat16)
    k = k_ref[0, 0].astype(jnp.bfloat16)
    v = v_ref[0, 0].astype(jnp.bfloat16)
    s = jax.lax.dot_general(
        q, k, (((1,), (1,)), ((), ())),
        preferred_element_type=jnp.float32)
    rows = jax.lax.broadcasted_iota(jnp.int32, s.shape, 0)
    cols = jax.lax.broadcasted_iota(jnp.int32, s.shape, 1)
    s = jnp.where(rows >= cols, s, jnp.float32(-1e9))
    m = jnp.max(s, axis=1, keepdims=True)
    e = jnp.exp(s - m)
    denom = jnp.sum(e, axis=1, keepdims=True)
    o = jnp.dot(e.astype(jnp.bfloat16), v,
                preferred_element_type=jnp.float32)
    o_ref[0, 0] = o / denom


def _attention(q, k, v):
    return pl.pallas_call(
        _attn_kernel,
        grid=(_B, _H),
        in_specs=[
            pl.BlockSpec((1, 1, _NM, _Dh), lambda b, h: (b, h, 0, 0)),
            pl.BlockSpec((1, 1, _NM, _Dh), lambda b, h: (b, h, 0, 0)),
            pl.BlockSpec((1, 1, _NM, _Dh), lambda b, h: (b, h, 0, 0)),
        ],
        out_specs=pl.BlockSpec((1, 1, _NM, _Dh), lambda b, h: (b, h, 0, 0)),
        out_shape=jax.ShapeDtypeStruct((_B, _H, _NM, _Dh), jnp.float32),
    )(q, k, v)


def _proj_kernel(a_ref, wo_ref, o_ref):
    acc = jnp.zeros((_NM, _C), dtype=jnp.float32)
    for h in range(_H):
        acc = acc + jnp.dot(
            a_ref[0, h], wo_ref[h * _Dh:(h + 1) * _Dh, :],
            preferred_element_type=jnp.float32)
    o_ref[0] = acc


def _out_proj(att, wo):
    return pl.pallas_call(
        _proj_kernel,
        grid=(_B,),
        in_specs=[
            pl.BlockSpec((1, _H, _NM, _Dh), lambda b: (b, 0, 0, 0)),
            pl.BlockSpec((_C, _C), lambda b: (0, 0)),
        ],
        out_specs=pl.BlockSpec((1, _NM, _C), lambda b: (b, 0, 0)),
        out_shape=jax.ShapeDtypeStruct((_B, _NM, _C), jnp.float32),
    )(att, wo)


_TPW = (_B * _N) // _NW
_TCH = _TPW // 2


@functools.lru_cache(maxsize=1)
def _sc_kernels():
    mesh = plsc.VectorSubcoreMesh(
        core_axis_name="c", subcore_axis_name="s", num_cores=_NC)

    @functools.partial(
        pl.kernel, mesh=mesh,
        out_type=jax.ShapeDtypeStruct((_B * _NM + _PAD, _C), jnp.float32),
        scratch_types=[
            pltpu.VMEM((_TCH,), jnp.int32),
            pltpu.VMEM((_TCH,), jnp.int32),
            pltpu.VMEM((_TCH, _C), jnp.float32),
            pltpu.VMEM((_TCH, _C), jnp.float32),
            pltpu.SemaphoreType.DMA,
            pltpu.SemaphoreType.DMA,
        ])
    def sc_merge(x_hbm, pf_hbm, dst_hbm, out_hbm, pf_v, dst_v, xr_v, pr_v,
                 s1, s2):
        wid = lax.axis_index("s") * _NC + lax.axis_index("c")
        base = wid * _TPW
        for c in range(_TPW // _TCH):
            co = base + c * _TCH
            pltpu.sync_copy(pf_hbm.at[pl.ds(co, _TCH)], pf_v)
            pltpu.sync_copy(dst_hbm.at[pl.ds(co, _TCH)], dst_v)
            cp1 = pltpu.async_copy(x_hbm.at[pl.ds(co, _TCH)], xr_v, s1)
            cp2 = pltpu.async_copy(x_hbm.at[pf_v], pr_v, s2)
            cp1.wait()
            cp2.wait()

            def row_body(i, _):
                def grp_body(j, _):
                    sl = pl.ds(j * 16, 16)
                    xr_v[i, sl] = (xr_v[i, sl] + pr_v[i, sl]) * 0.5
                    return 0
                return lax.fori_loop(0, _C // 16, grp_body, 0)

            lax.fori_loop(0, _TCH, row_body, 0)
            pltpu.async_copy(xr_v, out_hbm.at[dst_v], s1).wait()

    @functools.partial(
        pl.kernel, mesh=mesh,
        out_type=jax.ShapeDtypeStruct((_B * _N, _C), jnp.float32),
        scratch_types=[
            pltpu.VMEM((_UPW,), jnp.int32),
            pltpu.VMEM((_UPW, _C), jnp.float32),
            pltpu.SemaphoreType.DMA,
        ])
    def sc_unmerge(tbl_hbm, src_hbm, out_hbm, idx_v, rows_v, sem):
        wid = lax.axis_index("s") * _NC + lax.axis_index("c")
        base = wid * _UPW
        pltpu.sync_copy(src_hbm.at[pl.ds(base, _UPW)], idx_v)
        pltpu.async_copy(tbl_hbm.at[idx_v], rows_v, sem).wait()
        pltpu.sync_copy(rows_v, out_hbm.at[pl.ds(base, _UPW)])

    return sc_merge, sc_unmerge


def _sc_merge(xf, pf, dstf):
    return _sc_kernels()[0](xf, pf, dstf)


def _sc_unmerge(tbl, srcf):
    return _sc_kernels()[1](tbl, srcf)


def _build_e_mat():
    e = np.zeros((_Dh // 2, _HP * _Dh), dtype=np.float32)
    for c in range(_HP * _Dh):
        e[(c % _Dh) // 2, c] = 1.0
    return jnp.asarray(e)


def _build_s_mat():
    s = np.zeros((_HP * _Dh, _HP * _Dh), dtype=np.float32)
    for h in range(_HP):
        o = h * _Dh
        for i in range(_Dh // 2):
            s[o + 2 * i + 1, o + 2 * i] = -1.0
            s[o + 2 * i, o + 2 * i + 1] = 1.0
    return jnp.asarray(s)


@jax.jit
def _run(x, freqs_cis, wq, wk, wv, wo):
    bi = jnp.arange(_B, dtype=jnp.int32)[:, None]

    dst_e, dst_o, src_o, pe = _match(x.reshape(_B, _HALF, 2 * _C), wk)

    odd_ids = jnp.broadcast_to(
        (2 * jnp.arange(_HALF, dtype=jnp.int32) + 1)[None, :], (_B, _HALF))
    dst = jnp.stack([dst_e, dst_o], axis=-1).reshape(_B, _N)
    src = jnp.stack([dst_e, src_o], axis=-1).reshape(_B, _N)
    partner = jnp.stack([pe, odd_ids], axis=-1).reshape(_B, _N)

    trash = jnp.int32(_ZROW)
    dstf = jnp.where(dst < _NM, dst + bi * _NM, trash).reshape(-1)
    srcf = jnp.where(src < _NM, src + bi * _NM, trash).reshape(-1)
    pf = (partner + bi * _N).reshape(-1)

    x_mp = _sc_merge(x.reshape(_B * _N, _C), pf, dstf)

    cos = freqs_cis[:_NM, :, 0]
    sin = freqs_cis[:_NM, :, 1]
    e_mat = _build_e_mat()
    s_mat = _build_s_mat()

    q, k, v = _qkv(x_mp, wq, wk, wv, cos, sin, e_mat, s_mat)
    att = _attention(q, k, v)
    out_m = _out_proj(att, wo)

    tbl = jnp.concatenate(
        [out_m.reshape(_B * _NM, _C),
         jnp.zeros((_PAD, _C), jnp.float32)], axis=0)
    out = _sc_unmerge(tbl, srcf).reshape(_B, _N, _C)
    return out, k, v


def kernel(x, freqs_cis, Wq, Wk, Wv, Wo):
    return _run(x, freqs_cis, Wq, Wk, Wv, Wo)


# speedup vs baseline: 1.6104x; 1.0535x over previous
import functools

import jax
import jax.numpy as jnp
import numpy as np
from jax import lax
from jax.experimental import pallas as pl
from jax.experimental.pallas import tpu as pltpu
from jax.experimental.pallas import tpu_sc as plsc

_B, _N, _C = 2, 2048, 768
_H = 12
_Dh = _C // _H
_R = 256
_HALF = _N // 2
_NM = _N - _R

_NC = 2
_NS = 16
_NW = _NC * _NS
_MPW = (_B * _NM) // _NW
_MCH = _MPW // 2
_UPW = (_B * _N) // _NW
_PAD = 8
_ZROW = _B * _NM


def _scores_kernel(x2_ref, wk_ref, bs_ref, bb_ref):
    x2 = x2_ref[0]
    a = jnp.dot(x2[:, :_C], wk_ref[...], preferred_element_type=jnp.float32)
    b = jnp.dot(x2[:, _C:], wk_ref[...], preferred_element_type=jnp.float32)
    an = a / jnp.maximum(
        jnp.sqrt(jnp.sum(a * a, axis=1, keepdims=True)), 1e-12)
    bn = b / jnp.maximum(
        jnp.sqrt(jnp.sum(b * b, axis=1, keepdims=True)), 1e-12)
    scores = jax.lax.dot_general(
        an, bn, (((1,), (1,)), ((), ())),
        preferred_element_type=jnp.float32)
    ms = jnp.max(scores, axis=1, keepdims=True)
    cols = jax.lax.broadcasted_iota(jnp.int32, scores.shape, 1)
    cand = jnp.where(scores == ms, cols, jnp.int32(_HALF))
    bs_ref[0, 0, :] = ms[:, 0]
    bb_ref[0, 0, :] = jnp.min(cand, axis=1)


_SCH = 128


def _select_kernel(bs_ref, bb_ref, dste_ref, dsto_ref, srco_ref, pe_ref):
    bs = bs_ref[0, 0, :]
    bb = bb_ref[0, 0, :]
    bs_r = bs[None, :]
    bb_r = bb[None, :]
    col_i = jax.lax.broadcasted_iota(jnp.int32, (_SCH, _HALF), 1)
    nch = _HALF // _SCH

    ws_parts, wj_parts = [], []
    for c in range(nch):
        v_col = jax.lax.broadcasted_iota(
            jnp.int32, (_SCH, 1), 0) + jnp.int32(c * _SCH)
        m = bb_r == v_col
        ws_c = jnp.max(jnp.where(m, bs_r, jnp.float32(-2.0)), axis=1)
        wj_cand = jnp.where(
            jnp.logical_and(m, bs_r == ws_c[:, None]),
            col_i, jnp.int32(_HALF))
        ws_parts.append(ws_c)
        wj_parts.append(jnp.min(wj_cand, axis=1))
    ws = jnp.concatenate(ws_parts)
    wj = jnp.concatenate(wj_parts)
    valid = ws > jnp.float32(-1.5)

    ws_r = ws[None, :]
    wj_r = wj[None, :]
    valid_r = valid[None, :]
    rank_parts = []
    for c in range(nch):
        sl = slice(c * _SCH, (c + 1) * _SCH)
        ws_c = ws[sl][:, None]
        wj_c = wj[sl][:, None]
        better = jnp.logical_and(
            valid_r,
            jnp.logical_or(
                ws_r > ws_c,
                jnp.logical_and(ws_r == ws_c, wj_r < wj_c)))
        rank_parts.append(jnp.sum(better.astype(jnp.int32), axis=1))
    rank = jnp.concatenate(rank_parts)
    take = jnp.logical_and(valid, rank < _R)

    take8 = take.astype(jnp.float32).reshape(8, _HALF // 8)
    w = _HALF // 8
    ci = jax.lax.broadcasted_iota(jnp.int32, (w, w), 0)
    cj = jax.lax.broadcasted_iota(jnp.int32, (w, w), 1)
    u_incl = (ci <= cj).astype(jnp.float32)
    incl8 = jnp.dot(take8, u_incl, preferred_element_type=jnp.float32)
    rowsum = incl8[:, w - 1:w]
    ri = jax.lax.broadcasted_iota(jnp.int32, (8, 8), 0)
    rj = jax.lax.broadcasted_iota(jnp.int32, (8, 8), 1)
    s_strict = (rj < ri).astype(jnp.float32)
    rowoff = jnp.dot(s_strict, rowsum,
                     preferred_element_type=jnp.float32)
    tk = (incl8 + rowoff).reshape(_HALF) - take.astype(jnp.float32)

    v_iota_f = jax.lax.broadcasted_iota(
        jnp.int32, (_HALF,), 0).astype(jnp.float32)
    pos_even = 2.0 * v_iota_f - tk
    pos_odd = 2.0 * v_iota_f + 1.0 - tk

    wj_f = wj.astype(jnp.float32)
    take_f = take.astype(jnp.float32)
    wj_at_parts, tk_at_parts, posj_parts = [], [], []
    for c in range(nch):
        sl = slice(c * _SCH, (c + 1) * _SCH)
        a_uv = (bb[sl][:, None] == col_i).astype(jnp.float32)
        wj_at_parts.append(jnp.dot(a_uv, wj_f,
                                   preferred_element_type=jnp.float32))
        tk_at_parts.append(jnp.dot(a_uv, take_f,
                                   preferred_element_type=jnp.float32))
        o3 = (wj[sl][:, None] == col_i).astype(jnp.float32)
        posj_parts.append(jnp.dot(o3, pos_even,
                                  preferred_element_type=jnp.float32))
    wj_at_u = jnp.concatenate(wj_at_parts)
    tk_at_u = jnp.concatenate(tk_at_parts)
    posj = jnp.concatenate(posj_parts)

    u_iota_f = v_iota_f
    merged_a = jnp.logical_and(tk_at_u > 0.5, wj_at_u == u_iota_f)

    nm_f = jnp.float32(_NM)
    sent = jnp.int32(_NM)
    pos_even_ok = pos_even < nm_f
    pos_odd_ok = pos_odd < nm_f
    dst_e = jnp.where(pos_even_ok, pos_even.astype(jnp.int32), sent)
    keep_odd = jnp.logical_not(take)
    dst_o = jnp.where(jnp.logical_and(keep_odd, pos_odd_ok),
                      pos_odd.astype(jnp.int32), sent)
    src_o = jnp.where(
        take,
        jnp.where(posj < nm_f, posj.astype(jnp.int32), sent),
        jnp.where(pos_odd_ok, pos_odd.astype(jnp.int32), sent))
    v_iota_i = jax.lax.broadcasted_iota(jnp.int32, (_HALF,), 0)
    pe = jnp.where(merged_a, 2 * bb + 1, 2 * v_iota_i)

    dste_ref[0, 0, :] = dst_e
    dsto_ref[0, 0, :] = dst_o
    srco_ref[0, 0, :] = src_o
    pe_ref[0, 0, :] = pe


def _match(x2, wk):
    bs, bb = pl.pallas_call(
        _scores_kernel,
        grid=(_B,),
        in_specs=[
            pl.BlockSpec((1, _HALF, 2 * _C), lambda i: (i, 0, 0)),
            pl.BlockSpec((_C, _C), lambda i: (0, 0)),
        ],
        out_specs=[
            pl.BlockSpec((1, 1, _HALF), lambda i: (i, 0, 0)),
            pl.BlockSpec((1, 1, _HALF), lambda i: (i, 0, 0)),
        ],
        out_shape=[
            jax.ShapeDtypeStruct((_B, 1, _HALF), jnp.float32),
            jax.ShapeDtypeStruct((_B, 1, _HALF), jnp.int32),
        ],
    )(x2, wk)
    outs = pl.pallas_call(
        _select_kernel,
        grid=(_B,),
        in_specs=[
            pl.BlockSpec((1, 1, _HALF), lambda i: (i, 0, 0)),
            pl.BlockSpec((1, 1, _HALF), lambda i: (i, 0, 0)),
        ],
        out_specs=[
            pl.BlockSpec((1, 1, _HALF), lambda i: (i, 0, 0)),
            pl.BlockSpec((1, 1, _HALF), lambda i: (i, 0, 0)),
            pl.BlockSpec((1, 1, _HALF), lambda i: (i, 0, 0)),
            pl.BlockSpec((1, 1, _HALF), lambda i: (i, 0, 0)),
        ],
        out_shape=[
            jax.ShapeDtypeStruct((_B, 1, _HALF), jnp.int32),
            jax.ShapeDtypeStruct((_B, 1, _HALF), jnp.int32),
            jax.ShapeDtypeStruct((_B, 1, _HALF), jnp.int32),
            jax.ShapeDtypeStruct((_B, 1, _HALF), jnp.int32),
        ],
    )(bs, bb)
    return tuple(o[:, 0, :] for o in outs)


_HP = 2
_HB = _H // _HP


def _qkv_kernel(xm_ref, wq_ref, wk_ref, wv_ref, cos_ref, sin_ref, e_ref,
                s_ref, q_ref, k_ref, v_ref):
    x = xm_ref[...]
    e_mat = e_ref[...]
    cos = jnp.dot(cos_ref[...], e_mat, preferred_element_type=jnp.float32)
    sin = jnp.dot(sin_ref[...], e_mat, preferred_element_type=jnp.float32)
    s_mat = s_ref[...]

    q = jnp.dot(x, wq_ref[...], preferred_element_type=jnp.float32)
    q_rot = jnp.dot(q, s_mat, preferred_element_type=jnp.float32)
    qr = q * cos + q_rot * sin
    q_ref[0, 0] = qr[:, :_Dh]
    q_ref[0, 1] = qr[:, _Dh:]

    k = jnp.dot(x, wk_ref[...], preferred_element_type=jnp.float32)
    k_rot = jnp.dot(k, s_mat, preferred_element_type=jnp.float32)
    kr = k * cos + k_rot * sin
    k_ref[0, 0] = kr[:, :_Dh]
    k_ref[0, 1] = kr[:, _Dh:]

    v = jnp.dot(x, wv_ref[...], preferred_element_type=jnp.float32)
    v_ref[0, 0] = v[:, :_Dh]
    v_ref[0, 1] = v[:, _Dh:]


def _qkv(x_m, wq, wk, wv, cos_i, sin_i, e_mat, s_mat):
    wcols = _HP * _Dh
    return pl.pallas_call(
        _qkv_kernel,
        grid=(_B, _HB),
        in_specs=[
            pl.BlockSpec((_NM, _C), lambda b, h: (b, 0)),
            pl.BlockSpec((_C, wcols), lambda b, h: (0, h)),
            pl.BlockSpec((_C, wcols), lambda b, h: (0, h)),
            pl.BlockSpec((_C, wcols), lambda b, h: (0, h)),
            pl.BlockSpec((_NM, _Dh // 2), lambda b, h: (0, 0)),
            pl.BlockSpec((_NM, _Dh // 2), lambda b, h: (0, 0)),
            pl.BlockSpec((_Dh // 2, wcols), lambda b, h: (0, 0)),
            pl.BlockSpec((wcols, wcols), lambda b, h: (0, 0)),
        ],
        out_specs=[
            pl.BlockSpec((1, _HP, _NM, _Dh), lambda b, h: (b, h, 0, 0)),
            pl.BlockSpec((1, _HP, _NM, _Dh), lambda b, h: (b, h, 0, 0)),
            pl.BlockSpec((1, _HP, _NM, _Dh), lambda b, h: (b, h, 0, 0)),
        ],
        out_shape=[
            jax.ShapeDtypeStruct((_B, _H, _NM, _Dh), jnp.float32),
            jax.ShapeDtypeStruct((_B, _H, _NM, _Dh), jnp.float32),
            jax.ShapeDtypeStruct((_B, _H, _NM, _Dh), jnp.float32),
        ],
    )(x_m, wq, wk, wv, cos_i, sin_i, e_mat, s_mat)


def _attn_kernel(q_ref, k_ref, v_ref, o_ref):
    q = (q_ref[0, 0] * jnp.float32(1.0 / np.sqrt(_Dh))).astype(jnp.bfloat16)
    k = k_ref[0, 0].astype(jnp.bfloat16)
    v = v_ref[0, 0].astype(jnp.bfloat16)
    s = jax.lax.dot_general(
        q, k, (((1,), (1,)), ((), ())),
        preferred_element_type=jnp.float32)
    rows = jax.lax.broadcasted_iota(jnp.int32, s.shape, 0)
    cols = jax.lax.broadcasted_iota(jnp.int32, s.shape, 1)
    s = jnp.where(rows >= cols, s, jnp.float32(-1e9))
    m = jnp.max(s, axis=1, keepdims=True)
    e = jnp.exp(s - m)
    denom = jnp.sum(e, axis=1, keepdims=True)
    o = jnp.dot(e.astype(jnp.bfloat16), v,
                preferred_element_type=jnp.float32)
    o_ref[0, 0] = o / denom


def _attention(q, k, v):
    return pl.pallas_call(
        _attn_kernel,
        grid=(_B, _H),
        in_specs=[
            pl.BlockSpec((1, 1, _NM, _Dh), lambda b, h: (b, h, 0, 0)),
            pl.BlockSpec((1, 1, _NM, _Dh), lambda b, h: (b, h, 0, 0)),
            pl.BlockSpec((1, 1, _NM, _Dh), lambda b, h: (b, h, 0, 0)),
        ],
        out_specs=pl.BlockSpec((1, 1, _NM, _Dh), lambda b, h: (b, h, 0, 0)),
        out_shape=jax.ShapeDtypeStruct((_B, _H, _NM, _Dh), jnp.float32),
    )(q, k, v)


def _proj_kernel(a_ref, wo_ref, o_ref):
    acc = jnp.zeros((_NM, _C), dtype=jnp.float32)
    for h in range(_H):
        acc = acc + jnp.dot(
            a_ref[0, h], wo_ref[h * _Dh:(h + 1) * _Dh, :],
            preferred_element_type=jnp.float32)
    o_ref[0] = acc


def _out_proj(att, wo):
    return pl.pallas_call(
        _proj_kernel,
        grid=(_B,),
        in_specs=[
            pl.BlockSpec((1, _H, _NM, _Dh), lambda b: (b, 0, 0, 0)),
            pl.BlockSpec((_C, _C), lambda b: (0, 0)),
        ],
        out_specs=pl.BlockSpec((1, _NM, _C), lambda b: (b, 0, 0)),
        out_shape=jax.ShapeDtypeStruct((_B, _NM, _C), jnp.float32),
    )(att, wo)


_TPW = (_B * _N) // _NW
_TCH = _TPW // 2


@functools.lru_cache(maxsize=1)
def _sc_kernels():
    mesh = plsc.VectorSubcoreMesh(
        core_axis_name="c", subcore_axis_name="s", num_cores=_NC)

    @functools.partial(
        pl.kernel, mesh=mesh,
        out_type=jax.ShapeDtypeStruct((_B * _NM + _PAD, _C), jnp.float32),
        scratch_types=[
            pltpu.VMEM((_TCH,), jnp.int32),
            pltpu.VMEM((_TCH,), jnp.int32),
            pltpu.VMEM((_TCH, _C), jnp.float32),
            pltpu.VMEM((_TCH, _C), jnp.float32),
            pltpu.SemaphoreType.DMA,
            pltpu.SemaphoreType.DMA,
        ])
    def sc_merge(x_hbm, pf_hbm, dst_hbm, out_hbm, pf_v, dst_v, xr_v, pr_v,
                 s1, s2):
        wid = lax.axis_index("s") * _NC + lax.axis_index("c")
        base = wid * _TPW
        for c in range(_TPW // _TCH):
            co = base + c * _TCH
            pltpu.sync_copy(pf_hbm.at[pl.ds(co, _TCH)], pf_v)
            pltpu.sync_copy(dst_hbm.at[pl.ds(co, _TCH)], dst_v)
            cp1 = pltpu.async_copy(x_hbm.at[pl.ds(co, _TCH)], xr_v, s1)
            cp2 = pltpu.async_copy(x_hbm.at[pf_v], pr_v, s2)
            cp1.wait()
            cp2.wait()

            @plsc.parallel_loop(0, _TCH, step=1)
            def row_body(i):
                for j in range(_C // 16):
                    sl = pl.ds(j * 16, 16)
                    xr_v[i, sl] = (xr_v[i, sl] + pr_v[i, sl]) * 0.5
            pltpu.async_copy(xr_v, out_hbm.at[dst_v], s1).wait()

    @functools.partial(
        pl.kernel, mesh=mesh,
        out_type=jax.ShapeDtypeStruct((_B * _N, _C), jnp.float32),
        scratch_types=[
            pltpu.VMEM((_UPW,), jnp.int32),
            pltpu.VMEM((_UPW, _C), jnp.float32),
            pltpu.SemaphoreType.DMA,
        ])
    def sc_unmerge(tbl_hbm, src_hbm, out_hbm, idx_v, rows_v, sem):
        wid = lax.axis_index("s") * _NC + lax.axis_index("c")
        base = wid * _UPW
        pltpu.sync_copy(src_hbm.at[pl.ds(base, _UPW)], idx_v)
        pltpu.async_copy(tbl_hbm.at[idx_v], rows_v, sem).wait()
        pltpu.sync_copy(rows_v, out_hbm.at[pl.ds(base, _UPW)])

    return sc_merge, sc_unmerge


def _sc_merge(xf, pf, dstf):
    return _sc_kernels()[0](xf, pf, dstf)


def _sc_unmerge(tbl, srcf):
    return _sc_kernels()[1](tbl, srcf)


def _build_e_mat():
    e = np.zeros((_Dh // 2, _HP * _Dh), dtype=np.float32)
    for c in range(_HP * _Dh):
        e[(c % _Dh) // 2, c] = 1.0
    return jnp.asarray(e)


def _build_s_mat():
    s = np.zeros((_HP * _Dh, _HP * _Dh), dtype=np.float32)
    for h in range(_HP):
        o = h * _Dh
        for i in range(_Dh // 2):
            s[o + 2 * i + 1, o + 2 * i] = -1.0
            s[o + 2 * i, o + 2 * i + 1] = 1.0
    return jnp.asarray(s)


@jax.jit
def _run(x, freqs_cis, wq, wk, wv, wo):
    bi = jnp.arange(_B, dtype=jnp.int32)[:, None]

    dst_e, dst_o, src_o, pe = _match(x.reshape(_B, _HALF, 2 * _C), wk)

    odd_ids = jnp.broadcast_to(
        (2 * jnp.arange(_HALF, dtype=jnp.int32) + 1)[None, :], (_B, _HALF))
    dst = jnp.stack([dst_e, dst_o], axis=-1).reshape(_B, _N)
    src = jnp.stack([dst_e, src_o], axis=-1).reshape(_B, _N)
    partner = jnp.stack([pe, odd_ids], axis=-1).reshape(_B, _N)

    trash = jnp.int32(_ZROW)
    dstf = jnp.where(dst < _NM, dst + bi * _NM, trash).reshape(-1)
    srcf = jnp.where(src < _NM, src + bi * _NM, trash).reshape(-1)
    pf = (partner + bi * _N).reshape(-1)

    x_mp = _sc_merge(x.reshape(_B * _N, _C), pf, dstf)

    cos = freqs_cis[:_NM, :, 0]
    sin = freqs_cis[:_NM, :, 1]
    e_mat = _build_e_mat()
    s_mat = _build_s_mat()

    q, k, v = _qkv(x_mp, wq, wk, wv, cos, sin, e_mat, s_mat)
    att = _attention(q, k, v)
    out_m = _out_proj(att, wo)

    tbl = jnp.concatenate(
        [out_m.reshape(_B * _NM, _C),
         jnp.zeros((_PAD, _C), jnp.float32)], axis=0)
    out = _sc_unmerge(tbl, srcf).reshape(_B, _N, _C)
    return out, k, v


def kernel(x, freqs_cis, Wq, Wk, Wv, Wo):
    return _run(x, freqs_cis, Wq, Wk, Wv, Wo)


# speedup vs baseline: 1.6157x; 1.0033x over previous
import functools

import jax
import jax.numpy as jnp
import numpy as np
from jax import lax
from jax.experimental import pallas as pl
from jax.experimental.pallas import tpu as pltpu
from jax.experimental.pallas import tpu_sc as plsc

_B, _N, _C = 2, 2048, 768
_H = 12
_Dh = _C // _H
_R = 256
_HALF = _N // 2
_NM = _N - _R

_NC = 2
_NS = 16
_NW = _NC * _NS
_MPW = (_B * _NM) // _NW
_MCH = _MPW // 2
_UPW = (_B * _N) // _NW
_PAD = 8
_ZROW = _B * _NM


def _scores_kernel(x2_ref, wk_ref, bs_ref, bb_ref):
    x2 = x2_ref[0]
    a = jnp.dot(x2[:, :_C], wk_ref[...], preferred_element_type=jnp.float32)
    b = jnp.dot(x2[:, _C:], wk_ref[...], preferred_element_type=jnp.float32)
    an = a / jnp.maximum(
        jnp.sqrt(jnp.sum(a * a, axis=1, keepdims=True)), 1e-12)
    bn = b / jnp.maximum(
        jnp.sqrt(jnp.sum(b * b, axis=1, keepdims=True)), 1e-12)
    scores = jax.lax.dot_general(
        an, bn, (((1,), (1,)), ((), ())),
        preferred_element_type=jnp.float32)
    ms = jnp.max(scores, axis=1, keepdims=True)
    cols = jax.lax.broadcasted_iota(jnp.int32, scores.shape, 1)
    cand = jnp.where(scores == ms, cols, jnp.int32(_HALF))
    bs_ref[0, 0, :] = ms[:, 0]
    bb_ref[0, 0, :] = jnp.min(cand, axis=1)


_SCH = 128


def _select_kernel(bs_ref, bb_ref, dste_ref, dsto_ref, srco_ref, pe_ref):
    bs = bs_ref[0, 0, :]
    bb = bb_ref[0, 0, :]
    bs_r = bs[None, :]
    bb_r = bb[None, :]
    col_i = jax.lax.broadcasted_iota(jnp.int32, (_SCH, _HALF), 1)
    nch = _HALF // _SCH

    ws_parts, wj_parts = [], []
    for c in range(nch):
        v_col = jax.lax.broadcasted_iota(
            jnp.int32, (_SCH, 1), 0) + jnp.int32(c * _SCH)
        m = bb_r == v_col
        ws_c = jnp.max(jnp.where(m, bs_r, jnp.float32(-2.0)), axis=1)
        wj_cand = jnp.where(
            jnp.logical_and(m, bs_r == ws_c[:, None]),
            col_i, jnp.int32(_HALF))
        ws_parts.append(ws_c)
        wj_parts.append(jnp.min(wj_cand, axis=1))
    ws = jnp.concatenate(ws_parts)
    wj = jnp.concatenate(wj_parts)
    valid = ws > jnp.float32(-1.5)

    ws_r = ws[None, :]
    wj_r = wj[None, :]
    valid_r = valid[None, :]
    rank_parts = []
    for c in range(nch):
        sl = slice(c * _SCH, (c + 1) * _SCH)
        ws_c = ws[sl][:, None]
        wj_c = wj[sl][:, None]
        better = jnp.logical_and(
            valid_r,
            jnp.logical_or(
                ws_r > ws_c,
                jnp.logical_and(ws_r == ws_c, wj_r < wj_c)))
        rank_parts.append(jnp.sum(better.astype(jnp.int32), axis=1))
    rank = jnp.concatenate(rank_parts)
    take = jnp.logical_and(valid, rank < _R)

    take8 = take.astype(jnp.float32).reshape(8, _HALF // 8)
    w = _HALF // 8
    ci = jax.lax.broadcasted_iota(jnp.int32, (w, w), 0)
    cj = jax.lax.broadcasted_iota(jnp.int32, (w, w), 1)
    u_incl = (ci <= cj).astype(jnp.float32)
    incl8 = jnp.dot(take8, u_incl, preferred_element_type=jnp.float32)
    rowsum = incl8[:, w - 1:w]
    ri = jax.lax.broadcasted_iota(jnp.int32, (8, 8), 0)
    rj = jax.lax.broadcasted_iota(jnp.int32, (8, 8), 1)
    s_strict = (rj < ri).astype(jnp.float32)
    rowoff = jnp.dot(s_strict, rowsum,
                     preferred_element_type=jnp.float32)
    tk = (incl8 + rowoff).reshape(_HALF) - take.astype(jnp.float32)

    v_iota_f = jax.lax.broadcasted_iota(
        jnp.int32, (_HALF,), 0).astype(jnp.float32)
    pos_even = 2.0 * v_iota_f - tk
    pos_odd = 2.0 * v_iota_f + 1.0 - tk

    wj_f = wj.astype(jnp.float32)
    take_f = take.astype(jnp.float32)
    wj_at_parts, tk_at_parts, posj_parts = [], [], []
    for c in range(nch):
        sl = slice(c * _SCH, (c + 1) * _SCH)
        a_uv = (bb[sl][:, None] == col_i).astype(jnp.float32)
        wj_at_parts.append(jnp.dot(a_uv, wj_f,
                                   preferred_element_type=jnp.float32))
        tk_at_parts.append(jnp.dot(a_uv, take_f,
                                   preferred_element_type=jnp.float32))
        o3 = (wj[sl][:, None] == col_i).astype(jnp.float32)
        posj_parts.append(jnp.dot(o3, pos_even,
                                  preferred_element_type=jnp.float32))
    wj_at_u = jnp.concatenate(wj_at_parts)
    tk_at_u = jnp.concatenate(tk_at_parts)
    posj = jnp.concatenate(posj_parts)

    u_iota_f = v_iota_f
    merged_a = jnp.logical_and(tk_at_u > 0.5, wj_at_u == u_iota_f)

    nm_f = jnp.float32(_NM)
    sent = jnp.int32(_NM)
    pos_even_ok = pos_even < nm_f
    pos_odd_ok = pos_odd < nm_f
    dst_e = jnp.where(pos_even_ok, pos_even.astype(jnp.int32), sent)
    keep_odd = jnp.logical_not(take)
    dst_o = jnp.where(jnp.logical_and(keep_odd, pos_odd_ok),
                      pos_odd.astype(jnp.int32), sent)
    src_o = jnp.where(
        take,
        jnp.where(posj < nm_f, posj.astype(jnp.int32), sent),
        jnp.where(pos_odd_ok, pos_odd.astype(jnp.int32), sent))
    v_iota_i = jax.lax.broadcasted_iota(jnp.int32, (_HALF,), 0)
    pe = jnp.where(merged_a, 2 * bb + 1, 2 * v_iota_i)

    dste_ref[0, 0, :] = dst_e
    dsto_ref[0, 0, :] = dst_o
    srco_ref[0, 0, :] = src_o
    pe_ref[0, 0, :] = pe


def _match(x2, wk):
    bs, bb = pl.pallas_call(
        _scores_kernel,
        grid=(_B,),
        in_specs=[
            pl.BlockSpec((1, _HALF, 2 * _C), lambda i: (i, 0, 0)),
            pl.BlockSpec((_C, _C), lambda i: (0, 0)),
        ],
        out_specs=[
            pl.BlockSpec((1, 1, _HALF), lambda i: (i, 0, 0)),
            pl.BlockSpec((1, 1, _HALF), lambda i: (i, 0, 0)),
        ],
        out_shape=[
            jax.ShapeDtypeStruct((_B, 1, _HALF), jnp.float32),
            jax.ShapeDtypeStruct((_B, 1, _HALF), jnp.int32),
        ],
    )(x2, wk)
    outs = pl.pallas_call(
        _select_kernel,
        grid=(_B,),
        in_specs=[
            pl.BlockSpec((1, 1, _HALF), lambda i: (i, 0, 0)),
            pl.BlockSpec((1, 1, _HALF), lambda i: (i, 0, 0)),
        ],
        out_specs=[
            pl.BlockSpec((1, 1, _HALF), lambda i: (i, 0, 0)),
            pl.BlockSpec((1, 1, _HALF), lambda i: (i, 0, 0)),
            pl.BlockSpec((1, 1, _HALF), lambda i: (i, 0, 0)),
            pl.BlockSpec((1, 1, _HALF), lambda i: (i, 0, 0)),
        ],
        out_shape=[
            jax.ShapeDtypeStruct((_B, 1, _HALF), jnp.int32),
            jax.ShapeDtypeStruct((_B, 1, _HALF), jnp.int32),
            jax.ShapeDtypeStruct((_B, 1, _HALF), jnp.int32),
            jax.ShapeDtypeStruct((_B, 1, _HALF), jnp.int32),
        ],
    )(bs, bb)
    return tuple(o[:, 0, :] for o in outs)


_HP = 2
_HB = _H // _HP


def _qkv_kernel(xm_ref, wq_ref, wk_ref, wv_ref, cos_ref, sin_ref, e_ref,
                s_ref, q_ref, k_ref, v_ref):
    x = xm_ref[...].astype(jnp.bfloat16)
    e_mat = e_ref[...]
    cos = jnp.dot(cos_ref[...], e_mat, preferred_element_type=jnp.float32)
    sin = jnp.dot(sin_ref[...], e_mat, preferred_element_type=jnp.float32)
    s_mat = s_ref[...]

    q = jnp.dot(x, wq_ref[...].astype(jnp.bfloat16),
                preferred_element_type=jnp.float32)
    q_rot = jnp.dot(q, s_mat, preferred_element_type=jnp.float32)
    qr = q * cos + q_rot * sin
    q_ref[0, 0] = qr[:, :_Dh]
    q_ref[0, 1] = qr[:, _Dh:]

    k = jnp.dot(x, wk_ref[...].astype(jnp.bfloat16),
                preferred_element_type=jnp.float32)
    k_rot = jnp.dot(k, s_mat, preferred_element_type=jnp.float32)
    kr = k * cos + k_rot * sin
    k_ref[0, 0] = kr[:, :_Dh]
    k_ref[0, 1] = kr[:, _Dh:]

    v = jnp.dot(x, wv_ref[...].astype(jnp.bfloat16),
                preferred_element_type=jnp.float32)
    v_ref[0, 0] = v[:, :_Dh]
    v_ref[0, 1] = v[:, _Dh:]


def _qkv(x_m, wq, wk, wv, cos_i, sin_i, e_mat, s_mat):
    wcols = _HP * _Dh
    return pl.pallas_call(
        _qkv_kernel,
        grid=(_B, _HB),
        in_specs=[
            pl.BlockSpec((_NM, _C), lambda b, h: (b, 0)),
            pl.BlockSpec((_C, wcols), lambda b, h: (0, h)),
            pl.BlockSpec((_C, wcols), lambda b, h: (0, h)),
            pl.BlockSpec((_C, wcols), lambda b, h: (0, h)),
            pl.BlockSpec((_NM, _Dh // 2), lambda b, h: (0, 0)),
            pl.BlockSpec((_NM, _Dh // 2), lambda b, h: (0, 0)),
            pl.BlockSpec((_Dh // 2, wcols), lambda b, h: (0, 0)),
            pl.BlockSpec((wcols, wcols), lambda b, h: (0, 0)),
        ],
        out_specs=[
            pl.BlockSpec((1, _HP, _NM, _Dh), lambda b, h: (b, h, 0, 0)),
            pl.BlockSpec((1, _HP, _NM, _Dh), lambda b, h: (b, h, 0, 0)),
            pl.BlockSpec((1, _HP, _NM, _Dh), lambda b, h: (b, h, 0, 0)),
        ],
        out_shape=[
            jax.ShapeDtypeStruct((_B, _H, _NM, _Dh), jnp.float32),
            jax.ShapeDtypeStruct((_B, _H, _NM, _Dh), jnp.float32),
            jax.ShapeDtypeStruct((_B, _H, _NM, _Dh), jnp.float32),
        ],
    )(x_m, wq, wk, wv, cos_i, sin_i, e_mat, s_mat)


def _attn_kernel(q_ref, k_ref, v_ref, o_ref):
    q = (q_ref[0, 0] * jnp.float32(1.0 / np.sqrt(_Dh))).astype(jnp.bfloat16)
    k = k_ref[0, 0].astype(jnp.bfloat16)
    v = v_ref[0, 0].astype(jnp.bfloat16)
    s = jax.lax.dot_general(
        q, k, (((1,), (1,)), ((), ())),
        preferred_element_type=jnp.float32)
    rows = jax.lax.broadcasted_iota(jnp.int32, s.shape, 0)
    cols = jax.lax.broadcasted_iota(jnp.int32, s.shape, 1)
    s = jnp.where(rows >= cols, s, jnp.float32(-1e9))
    m = jnp.max(s, axis=1, keepdims=True)
    e = jnp.exp(s - m)
    denom = jnp.sum(e, axis=1, keepdims=True)
    o = jnp.dot(e.astype(jnp.bfloat16), v,
                preferred_element_type=jnp.float32)
    o_ref[0, 0] = o / denom


def _attention(q, k, v):
    return pl.pallas_call(
        _attn_kernel,
        grid=(_B, _H),
        in_specs=[
            pl.BlockSpec((1, 1, _NM, _Dh), lambda b, h: (b, h, 0, 0)),
            pl.BlockSpec((1, 1, _NM, _Dh), lambda b, h: (b, h, 0, 0)),
            pl.BlockSpec((1, 1, _NM, _Dh), lambda b, h: (b, h, 0, 0)),
        ],
        out_specs=pl.BlockSpec((1, 1, _NM, _Dh), lambda b, h: (b, h, 0, 0)),
        out_shape=jax.ShapeDtypeStruct((_B, _H, _NM, _Dh), jnp.float32),
    )(q, k, v)


def _proj_kernel(a_ref, wo_ref, o_ref):
    acc = jnp.zeros((_NM, _C), dtype=jnp.float32)
    for h in range(_H):
        acc = acc + jnp.dot(
            a_ref[0, h], wo_ref[h * _Dh:(h + 1) * _Dh, :],
            preferred_element_type=jnp.float32)
    o_ref[0] = acc


def _out_proj(att, wo):
    return pl.pallas_call(
        _proj_kernel,
        grid=(_B,),
        in_specs=[
            pl.BlockSpec((1, _H, _NM, _Dh), lambda b: (b, 0, 0, 0)),
            pl.BlockSpec((_C, _C), lambda b: (0, 0)),
        ],
        out_specs=pl.BlockSpec((1, _NM, _C), lambda b: (b, 0, 0)),
        out_shape=jax.ShapeDtypeStruct((_B, _NM, _C), jnp.float32),
    )(att, wo)


_TPW = (_B * _N) // _NW
_TCH = _TPW // 2


@functools.lru_cache(maxsize=1)
def _sc_kernels():
    mesh = plsc.VectorSubcoreMesh(
        core_axis_name="c", subcore_axis_name="s", num_cores=_NC)

    @functools.partial(
        pl.kernel, mesh=mesh,
        out_type=jax.ShapeDtypeStruct((_B * _NM + _PAD, _C), jnp.float32),
        scratch_types=[
            pltpu.VMEM((_TCH,), jnp.int32),
            pltpu.VMEM((_TCH,), jnp.int32),
            pltpu.VMEM((_TCH, _C), jnp.float32),
            pltpu.VMEM((_TCH, _C), jnp.float32),
            pltpu.SemaphoreType.DMA,
            pltpu.SemaphoreType.DMA,
        ])
    def sc_merge(x_hbm, pf_hbm, dst_hbm, out_hbm, pf_v, dst_v, xr_v, pr_v,
                 s1, s2):
        wid = lax.axis_index("s") * _NC + lax.axis_index("c")
        base = wid * _TPW
        for c in range(_TPW // _TCH):
            co = base + c * _TCH
            pltpu.sync_copy(pf_hbm.at[pl.ds(co, _TCH)], pf_v)
            pltpu.sync_copy(dst_hbm.at[pl.ds(co, _TCH)], dst_v)
            cp1 = pltpu.async_copy(x_hbm.at[pl.ds(co, _TCH)], xr_v, s1)
            cp2 = pltpu.async_copy(x_hbm.at[pf_v], pr_v, s2)
            cp1.wait()
            cp2.wait()

            @plsc.parallel_loop(0, _TCH, step=1)
            def row_body(i):
                for j in range(_C // 16):
                    sl = pl.ds(j * 16, 16)
                    xr_v[i, sl] = (xr_v[i, sl] + pr_v[i, sl]) * 0.5
            pltpu.async_copy(xr_v, out_hbm.at[dst_v], s1).wait()

    @functools.partial(
        pl.kernel, mesh=mesh,
        out_type=jax.ShapeDtypeStruct((_B * _N, _C), jnp.float32),
        scratch_types=[
            pltpu.VMEM((_UPW,), jnp.int32),
            pltpu.VMEM((_UPW, _C), jnp.float32),
            pltpu.SemaphoreType.DMA,
        ])
    def sc_unmerge(tbl_hbm, src_hbm, out_hbm, idx_v, rows_v, sem):
        wid = lax.axis_index("s") * _NC + lax.axis_index("c")
        base = wid * _UPW
        pltpu.sync_copy(src_hbm.at[pl.ds(base, _UPW)], idx_v)
        pltpu.async_copy(tbl_hbm.at[idx_v], rows_v, sem).wait()
        pltpu.sync_copy(rows_v, out_hbm.at[pl.ds(base, _UPW)])

    return sc_merge, sc_unmerge


def _sc_merge(xf, pf, dstf):
    return _sc_kernels()[0](xf, pf, dstf)


def _sc_unmerge(tbl, srcf):
    return _sc_kernels()[1](tbl, srcf)


def _build_e_mat():
    e = np.zeros((_Dh // 2, _HP * _Dh), dtype=np.float32)
    for c in range(_HP * _Dh):
        e[(c % _Dh) // 2, c] = 1.0
    return jnp.asarray(e)


def _build_s_mat():
    s = np.zeros((_HP * _Dh, _HP * _Dh), dtype=np.float32)
    for h in range(_HP):
        o = h * _Dh
        for i in range(_Dh // 2):
            s[o + 2 * i + 1, o + 2 * i] = -1.0
            s[o + 2 * i, o + 2 * i + 1] = 1.0
    return jnp.asarray(s)


@jax.jit
def _run(x, freqs_cis, wq, wk, wv, wo):
    bi = jnp.arange(_B, dtype=jnp.int32)[:, None]

    dst_e, dst_o, src_o, pe = _match(x.reshape(_B, _HALF, 2 * _C), wk)

    odd_ids = jnp.broadcast_to(
        (2 * jnp.arange(_HALF, dtype=jnp.int32) + 1)[None, :], (_B, _HALF))
    dst = jnp.stack([dst_e, dst_o], axis=-1).reshape(_B, _N)
    src = jnp.stack([dst_e, src_o], axis=-1).reshape(_B, _N)
    partner = jnp.stack([pe, odd_ids], axis=-1).reshape(_B, _N)

    trash = jnp.int32(_ZROW)
    dstf = jnp.where(dst < _NM, dst + bi * _NM, trash).reshape(-1)
    srcf = jnp.where(src < _NM, src + bi * _NM, trash).reshape(-1)
    pf = (partner + bi * _N).reshape(-1)

    x_mp = _sc_merge(x.reshape(_B * _N, _C), pf, dstf)

    cos = freqs_cis[:_NM, :, 0]
    sin = freqs_cis[:_NM, :, 1]
    e_mat = _build_e_mat()
    s_mat = _build_s_mat()

    q, k, v = _qkv(x_mp, wq, wk, wv, cos, sin, e_mat, s_mat)
    att = _attention(q, k, v)
    out_m = _out_proj(att, wo)

    tbl = jnp.concatenate(
        [out_m.reshape(_B * _NM, _C),
         jnp.zeros((_PAD, _C), jnp.float32)], axis=0)
    out = _sc_unmerge(tbl, srcf).reshape(_B, _N, _C)
    return out, k, v


def kernel(x, freqs_cis, Wq, Wk, Wv, Wo):
    return _run(x, freqs_cis, Wq, Wk, Wv, Wo)
